# Initial kernel scaffold; baseline (speedup 1.0000x reference)
import jax
import jax.numpy as jnp
from jax.experimental import pallas as pl


def kernel(x, edge_index, a1, bn_w, bn_b, W1, b1, a2, W2, b2):
    raise NotImplementedError("write your pallas kernel here")


# speedup vs baseline: 12.1471x; 12.1471x over previous
import jax
import jax.numpy as jnp
from jax import lax
from jax.experimental import pallas as pl
from jax.experimental.pallas import tpu as pltpu
from jax.experimental.pallas import tpu_sc as plsc

N = 10000
C = 128
E = 320000
NC = 2
NS = 16
NW = NC * NS
EPW = E // NW
K = 80
RPT = 624
ZR = 104

def _deg_body(dst_hbm, out_hbm, acc, idx_v, ones_v, zb):
    c = lax.axis_index("c")
    s = lax.axis_index("s")

    @pl.loop(0, K, step=16)
    def _(i):
        ones_v.at[pl.ds(i, 16)][...] = jnp.ones((16,), jnp.float32)

    @pl.loop(0, 2000, step=16)
    def _(i):
        zb.at[pl.ds(i, 16)][...] = jnp.zeros((16,), jnp.float32)

    @pl.when(s < 5)
    def _():
        pltpu.sync_copy(zb, acc.at[pl.ds(s * 2000, 2000)])

    plsc.subcore_barrier()

    base = (c * NS + s) * EPW

    @pl.loop(0, EPW, step=K)
    def _(j):
        pltpu.sync_copy(dst_hbm.at[pl.ds(base + j, K)], idx_v)
        pltpu.sync_copy(ones_v, acc.at[idx_v], add=True)

    plsc.subcore_barrier()

    @pl.when(s < 10)
    def _():
        pltpu.sync_copy(acc.at[pl.ds(s * 1000, 1000)], zb.at[pl.ds(0, 1000)])
        pltpu.sync_copy(zb.at[pl.ds(0, 1000)],
                        out_hbm.at[pl.ds(c * N + s * 1000, 1000)])


_sc_calls = {}


def _get_sc_calls():
    if not _sc_calls:
        mesh = plsc.VectorSubcoreMesh(core_axis_name="c", subcore_axis_name="s",
                                      num_cores=NC, num_subcores=NS)
        _sc_calls["deg"] = pl.kernel(
            _deg_body,
            out_type=jax.ShapeDtypeStruct((NC * N,), jnp.float32),
            mesh=mesh,
            scratch_types=[
                pltpu.VMEM_SHARED((N,), jnp.float32),
                pltpu.VMEM((K,), jnp.int32),
                pltpu.VMEM((K,), jnp.float32),
                pltpu.VMEM((2000,), jnp.float32),
            ],
        )
        _sc_calls["agg"] = pl.kernel(
            _agg_body,
            out_type=jax.ShapeDtypeStruct((NC, N, C), jnp.float32),
            mesh=mesh,
            scratch_types=[
                pltpu.VMEM_SHARED((N, C), jnp.float32),
                pltpu.VMEM((K,), jnp.int32),
                pltpu.VMEM((K,), jnp.int32),
                pltpu.VMEM((K, C), jnp.float32),
                pltpu.VMEM((ZR, C), jnp.float32),
            ],
        )
    return _sc_calls["deg"], _sc_calls["agg"]


def _agg_body(u_hbm, src_hbm, dst_hbm, out_hbm, acc, sidx, didx, rows, zb):
    c = lax.axis_index("c")
    s = lax.axis_index("s")

    @pl.loop(0, ZR)
    def _(i):
        @pl.loop(0, C, step=16)
        def _(k):
            zb.at[pl.ds(i, 1), pl.ds(k, 16)][...] = jnp.zeros((1, 16), jnp.float32)

    @pl.loop(0, RPT, step=ZR)
    def _(r):
        pltpu.sync_copy(zb, acc.at[pl.ds(s * RPT + r, ZR)])

    @pl.when(s == NS - 1)
    def _():
        pltpu.sync_copy(zb.at[pl.ds(0, 16)], acc.at[pl.ds(NS * RPT, 16)])

    plsc.subcore_barrier()

    base = (c * NS + s) * EPW

    @pl.loop(0, EPW, step=K)
    def _(j):
        pltpu.sync_copy(src_hbm.at[pl.ds(base + j, K)], sidx)
        pltpu.sync_copy(dst_hbm.at[pl.ds(base + j, K)], didx)
        pltpu.sync_copy(u_hbm.at[sidx], rows)
        pltpu.sync_copy(rows, acc.at[didx], add=True)

    plsc.subcore_barrier()

    @pl.loop(0, RPT, step=ZR)
    def _(r):
        pltpu.sync_copy(acc.at[pl.ds(s * RPT + r, ZR)], zb)
        pltpu.sync_copy(zb, out_hbm.at[c, pl.ds(s * RPT + r, ZR)])

    @pl.when(s == NS - 1)
    def _():
        pltpu.sync_copy(acc.at[pl.ds(NS * RPT, 16)], zb.at[pl.ds(0, 16)])
        pltpu.sync_copy(zb.at[pl.ds(0, 16)], out_hbm.at[c, pl.ds(NS * RPT, 16)])


def _tc1_body(x_ref, degp_ref, a1_ref, bnw_ref, bnb_ref, w1_ref, u1_ref, dinv_ref):
    x = x_ref[...]
    h = jnp.where(x >= 0, x, a1_ref[...] * x)
    mean = jnp.mean(h, axis=0, keepdims=True)
    var = jnp.mean(jnp.square(h - mean), axis=0, keepdims=True)
    h = (h - mean) * lax.rsqrt(var + 1e-5) * bnw_ref[...] + bnb_ref[...]
    deg = degp_ref[0] + degp_ref[1] + 1.0
    dinv = lax.rsqrt(deg)
    ht = jnp.dot(h, w1_ref[...], preferred_element_type=jnp.float32)
    u1_ref[...] = ht * dinv
    dinv_ref[...] = dinv


def _tc2_body(p_ref, u1_ref, dinv_ref, a2_ref, b1_ref, w2_ref, u2_ref):
    dinv = dinv_ref[...]
    agg = (p_ref[0] + p_ref[1] + u1_ref[...]) * dinv + b1_ref[...]
    h2 = jnp.where(agg >= 0, agg, a2_ref[...] * agg)
    u2_ref[...] = jnp.dot(h2, w2_ref[...], preferred_element_type=jnp.float32) * dinv


def _tc3_body(p_ref, u2_ref, dinv_ref, b2_ref, out_ref):
    out_ref[...] = (p_ref[0] + p_ref[1] + u2_ref[...]) * dinv_ref[...] + b2_ref[...]


_tc1 = pl.pallas_call(
    _tc1_body,
    out_shape=(jax.ShapeDtypeStruct((N, C), jnp.float32),
               jax.ShapeDtypeStruct((N, 1), jnp.float32)),
)

_tc2 = pl.pallas_call(
    _tc2_body,
    out_shape=jax.ShapeDtypeStruct((N, C), jnp.float32),
)

_tc3 = pl.pallas_call(
    _tc3_body,
    out_shape=jax.ShapeDtypeStruct((N, C), jnp.float32),
)


def kernel(x, edge_index, a1, bn_w, bn_b, W1, b1, a2, W2, b2):
    src = edge_index[0]
    dst = edge_index[1]
    _deg_call, _agg_call = _get_sc_calls()
    degp = _deg_call(dst)
    degp3 = degp.reshape(NC, N, 1)
    u1, dinv = _tc1(x, degp3, a1.reshape(1, 1), bn_w.reshape(1, C),
                    bn_b.reshape(1, C), W1)
    p1 = _agg_call(u1, src, dst)
    u2 = _tc2(p1, u1, dinv, a2.reshape(1, 1), b1.reshape(1, C), W2)
    p2 = _agg_call(u2, src, dst)
    out = _tc3(p2, u2, dinv, b2.reshape(1, C))
    return out


# speedup vs baseline: 26.8724x; 2.2122x over previous
import jax
import jax.numpy as jnp
from jax import lax
from jax.experimental import pallas as pl
from jax.experimental.pallas import tpu as pltpu
from jax.experimental.pallas import tpu_sc as plsc

N = 10000
C = 128
E = 320000
NC = 2
NS = 16
NW = NC * NS
EPW = E // NW
K = 80
NCH = EPW // K
NB = 2
RPT = 624
ZR = 48


def _deg_body(dst_hbm, out_hbm, acc, didx, ones_v, zbd, dsem):
    c = lax.axis_index("c")
    s = lax.axis_index("s")

    @pl.loop(0, K, step=16)
    def _(i):
        ones_v.at[pl.ds(i, 16)][...] = jnp.ones((16,), jnp.float32)

    @pl.loop(0, 2000, step=16)
    def _(i):
        zbd.at[pl.ds(i, 16)][...] = jnp.zeros((16,), jnp.float32)

    @pl.when(s < 5)
    def _():
        pltpu.sync_copy(zbd, acc.at[pl.ds(s * 2000, 2000)])

    plsc.subcore_barrier()

    base = (c * NS + s) * EPW

    def _fire(j):
        pltpu.async_copy(dst_hbm.at[pl.ds(base + j * K, K)],
                         didx.at[lax.rem(j, NB)], dsem)

    def _wait(j):
        pltpu.make_async_copy(dst_hbm.at[pl.ds(base + j * K, K)],
                              didx.at[lax.rem(j, NB)], dsem).wait()

    _fire(0)
    _fire(1)

    @pl.loop(0, NCH)
    def _(j):
        _wait(j)
        pltpu.sync_copy(ones_v, acc.at[didx.at[lax.rem(j, NB)]], add=True)

        @pl.when(j + NB < NCH)
        def _():
            _fire(j + NB)

    plsc.subcore_barrier()

    @pl.when(s < 10)
    def _():
        pltpu.sync_copy(acc.at[pl.ds(s * 1000, 1000)], zbd.at[pl.ds(0, 1000)])
        pltpu.sync_copy(zbd.at[pl.ds(0, 1000)],
                        out_hbm.at[pl.ds(c * N + s * 1000, 1000)])


def _agg_body(u_hbm, src_hbm, dst_hbm, out_hbm, acc, sidx_all, didx, rows,
              zb0, zb1, gsem, dsem, osem):
    c = lax.axis_index("c")
    s = lax.axis_index("s")

    @pl.loop(0, ZR)
    def _(i):
        @pl.loop(0, C, step=16)
        def _(k):
            zb0.at[pl.ds(i, 1), pl.ds(k, 16)][...] = jnp.zeros((1, 16), jnp.float32)

    for r in range(0, RPT, ZR):
        pltpu.async_copy(zb0, acc.at[pl.ds(s * RPT + r, ZR)], osem)

    @pl.when(s == NS - 1)
    def _():
        pltpu.async_copy(zb0.at[pl.ds(0, 16)], acc.at[pl.ds(NS * RPT, 16)], osem)

    for r in range(0, RPT, ZR):
        pltpu.make_async_copy(zb0, acc.at[pl.ds(s * RPT + r, ZR)], osem).wait()

    @pl.when(s == NS - 1)
    def _():
        pltpu.make_async_copy(zb0.at[pl.ds(0, 16)], acc.at[pl.ds(NS * RPT, 16)],
                              osem).wait()

    base = (c * NS + s) * EPW
    pltpu.sync_copy(src_hbm.at[pl.ds(base, EPW)], sidx_all)

    plsc.subcore_barrier()

    def _fire(j):
        b = lax.rem(j, NB)
        pltpu.async_copy(dst_hbm.at[pl.ds(base + j * K, K)], didx.at[b], dsem)
        pltpu.async_copy(u_hbm.at[sidx_all.at[pl.ds(j * K, K)]], rows.at[b], gsem)

    def _wait(j):
        b = lax.rem(j, NB)
        pltpu.make_async_copy(dst_hbm.at[pl.ds(base + j * K, K)], didx.at[b],
                              dsem).wait()
        pltpu.make_async_copy(u_hbm.at[sidx_all.at[pl.ds(j * K, K)]], rows.at[b],
                              gsem).wait()

    _fire(0)
    _fire(1)

    @pl.loop(0, NCH)
    def _(j):
        _wait(j)
        b = lax.rem(j, NB)
        pltpu.sync_copy(rows.at[b], acc.at[didx.at[b]], add=True)

        @pl.when(j + NB < NCH)
        def _():
            _fire(j + NB)

    plsc.subcore_barrier()

    zbufs = (zb0, zb1)
    nch_out = RPT // ZR
    for r_i in range(nch_out):
        buf = zbufs[r_i % 2]
        lo = s * RPT + r_i * ZR
        if r_i >= 2:
            plo = s * RPT + (r_i - 2) * ZR
            pltpu.make_async_copy(buf, out_hbm.at[c, pl.ds(plo, ZR)], osem).wait()
        pltpu.sync_copy(acc.at[pl.ds(lo, ZR)], buf)
        pltpu.async_copy(buf, out_hbm.at[c, pl.ds(lo, ZR)], osem)
    for r_i in range(nch_out - 2, nch_out):
        buf = zbufs[r_i % 2]
        lo = s * RPT + r_i * ZR
        pltpu.make_async_copy(buf, out_hbm.at[c, pl.ds(lo, ZR)], osem).wait()

    @pl.when(s == NS - 1)
    def _():
        pltpu.sync_copy(acc.at[pl.ds(NS * RPT, 16)], zb0.at[pl.ds(0, 16)])
        pltpu.sync_copy(zb0.at[pl.ds(0, 16)], out_hbm.at[c, pl.ds(NS * RPT, 16)])


_sc_calls = {}


def _get_sc_calls():
    if not _sc_calls:
        mesh = plsc.VectorSubcoreMesh(core_axis_name="c", subcore_axis_name="s",
                                      num_cores=NC, num_subcores=NS)
        _sc_calls["deg"] = pl.kernel(
            _deg_body,
            out_type=jax.ShapeDtypeStruct((NC * N,), jnp.float32),
            mesh=mesh,
            scratch_types=[
                pltpu.VMEM_SHARED((N,), jnp.float32),
                pltpu.VMEM((NB, K), jnp.int32),
                pltpu.VMEM((K,), jnp.float32),
                pltpu.VMEM((2000,), jnp.float32),
                pltpu.SemaphoreType.DMA,
            ],
        )
        _sc_calls["agg"] = pl.kernel(
            _agg_body,
            out_type=jax.ShapeDtypeStruct((NC, N, C), jnp.float32),
            mesh=mesh,
            scratch_types=[
                pltpu.VMEM_SHARED((N, C), jnp.float32),
                pltpu.VMEM((EPW,), jnp.int32),
                pltpu.VMEM((NB, K), jnp.int32),
                pltpu.VMEM((NB, K, C), jnp.float32),
                pltpu.VMEM((ZR, C), jnp.float32),
                pltpu.VMEM((ZR, C), jnp.float32),
                pltpu.SemaphoreType.DMA,
                pltpu.SemaphoreType.DMA,
                pltpu.SemaphoreType.DMA,
            ],
        )
    return _sc_calls["deg"], _sc_calls["agg"]


def _tc1_body(x_ref, degp_ref, a1_ref, bnw_ref, bnb_ref, w1_ref, u1_ref, dinv_ref):
    x = x_ref[...]
    h = jnp.where(x >= 0, x, a1_ref[...] * x)
    mean = jnp.mean(h, axis=0, keepdims=True)
    var = jnp.mean(jnp.square(h - mean), axis=0, keepdims=True)
    h = (h - mean) * lax.rsqrt(var + 1e-5) * bnw_ref[...] + bnb_ref[...]
    deg = degp_ref[0] + degp_ref[1] + 1.0
    dinv = lax.rsqrt(deg)
    ht = jnp.dot(h, w1_ref[...], preferred_element_type=jnp.float32)
    u1_ref[...] = ht * dinv
    dinv_ref[...] = dinv


def _tc2_body(p_ref, u1_ref, dinv_ref, a2_ref, b1_ref, w2_ref, u2_ref):
    dinv = dinv_ref[...]
    agg = (p_ref[0] + p_ref[1] + u1_ref[...]) * dinv + b1_ref[...]
    h2 = jnp.where(agg >= 0, agg, a2_ref[...] * agg)
    u2_ref[...] = jnp.dot(h2, w2_ref[...], preferred_element_type=jnp.float32) * dinv


def _tc3_body(p_ref, u2_ref, dinv_ref, b2_ref, out_ref):
    out_ref[...] = (p_ref[0] + p_ref[1] + u2_ref[...]) * dinv_ref[...] + b2_ref[...]


_tc1 = pl.pallas_call(
    _tc1_body,
    out_shape=(jax.ShapeDtypeStruct((N, C), jnp.float32),
               jax.ShapeDtypeStruct((N, 1), jnp.float32)),
)

_tc2 = pl.pallas_call(
    _tc2_body,
    out_shape=jax.ShapeDtypeStruct((N, C), jnp.float32),
)

_tc3 = pl.pallas_call(
    _tc3_body,
    out_shape=jax.ShapeDtypeStruct((N, C), jnp.float32),
)


def kernel(x, edge_index, a1, bn_w, bn_b, W1, b1, a2, W2, b2):
    src = edge_index[0]
    dst = edge_index[1]
    _deg_call, _agg_call = _get_sc_calls()
    degp = _deg_call(dst)
    degp3 = degp.reshape(NC, N, 1)
    u1, dinv = _tc1(x, degp3, a1.reshape(1, 1), bn_w.reshape(1, C),
                    bn_b.reshape(1, C), W1)
    p1 = _agg_call(u1, src, dst)
    u2 = _tc2(p1, u1, dinv, a2.reshape(1, 1), b1.reshape(1, C), W2)
    p2 = _agg_call(u2, src, dst)
    out = _tc3(p2, u2, dinv, b2.reshape(1, C))
    return out


# speedup vs baseline: 30.6370x; 1.1401x over previous
import jax
import jax.numpy as jnp
from jax import lax
from jax.experimental import pallas as pl
from jax.experimental.pallas import tpu as pltpu
from jax.experimental.pallas import tpu_sc as plsc

N = 10000
C = 128
E = 320000
NC = 2
NS = 16
NW = NC * NS
EPW = E // NW
K = 80
NCH = EPW // K
NB = 3
RPT = 624
ZR = 24


def _deg_body(dst_hbm, out_hbm, acc, didx, ones_v, zbd, dsem, ssem):
    c = lax.axis_index("c")
    s = lax.axis_index("s")

    @pl.loop(0, K, step=16)
    def _(i):
        ones_v.at[pl.ds(i, 16)][...] = jnp.ones((16,), jnp.float32)

    @pl.loop(0, 2000, step=16)
    def _(i):
        zbd.at[pl.ds(i, 16)][...] = jnp.zeros((16,), jnp.float32)

    @pl.when(s < 5)
    def _():
        pltpu.sync_copy(zbd, acc.at[pl.ds(s * 2000, 2000)])

    plsc.subcore_barrier()

    base = (c * NS + s) * EPW

    def _fire(j):
        pltpu.async_copy(dst_hbm.at[pl.ds(base + j * K, K)],
                         didx.at[lax.rem(j, NB)], dsem)

    def _wait(j):
        pltpu.make_async_copy(dst_hbm.at[pl.ds(base + j * K, K)],
                              didx.at[lax.rem(j, NB)], dsem).wait()

    def _fire_s(j):
        pltpu.async_copy(ones_v, acc.at[didx.at[lax.rem(j, NB)]], ssem, add=True)

    def _wait_s(j):
        pltpu.make_async_copy(ones_v, acc.at[didx.at[lax.rem(j, NB)]],
                              ssem).wait()

    _fire(0)
    _fire(1)

    @pl.loop(0, NCH)
    def _(j):
        _wait(j)
        _fire_s(j)

        @pl.when(j >= 1)
        def _():
            _wait_s(j - 1)

        @pl.when(j + 2 < NCH)
        def _():
            _fire(j + 2)

    _wait_s(NCH - 1)

    plsc.subcore_barrier()

    @pl.when(s < 10)
    def _():
        pltpu.sync_copy(acc.at[pl.ds(s * 1000, 1000)], zbd.at[pl.ds(0, 1000)])
        pltpu.sync_copy(zbd.at[pl.ds(0, 1000)],
                        out_hbm.at[pl.ds(c * N + s * 1000, 1000)])


def _agg_body(u_hbm, src_hbm, dst_hbm, out_hbm, acc, sidx_all, didx, rows,
              zb0, zb1, gsem, dsem, osem, ssem):
    c = lax.axis_index("c")
    s = lax.axis_index("s")

    @pl.loop(0, ZR)
    def _(i):
        @pl.loop(0, C, step=16)
        def _(k):
            zb0.at[pl.ds(i, 1), pl.ds(k, 16)][...] = jnp.zeros((1, 16), jnp.float32)

    for r in range(0, RPT, ZR):
        pltpu.async_copy(zb0, acc.at[pl.ds(s * RPT + r, ZR)], osem)

    @pl.when(s == NS - 1)
    def _():
        pltpu.async_copy(zb0.at[pl.ds(0, 16)], acc.at[pl.ds(NS * RPT, 16)], osem)

    base = (c * NS + s) * EPW
    pltpu.sync_copy(src_hbm.at[pl.ds(base, EPW)], sidx_all)

    def _fire(j):
        b = lax.rem(j, NB)
        pltpu.async_copy(dst_hbm.at[pl.ds(base + j * K, K)], didx.at[b], dsem)
        pltpu.async_copy(u_hbm.at[sidx_all.at[pl.ds(j * K, K)]], rows.at[b], gsem)

    def _wait(j):
        b = lax.rem(j, NB)
        pltpu.make_async_copy(dst_hbm.at[pl.ds(base + j * K, K)], didx.at[b],
                              dsem).wait()
        pltpu.make_async_copy(u_hbm.at[sidx_all.at[pl.ds(j * K, K)]], rows.at[b],
                              gsem).wait()

    def _fire_s(j):
        b = lax.rem(j, NB)
        pltpu.async_copy(rows.at[b], acc.at[didx.at[b]], ssem, add=True)

    def _wait_s(j):
        b = lax.rem(j, NB)
        pltpu.make_async_copy(rows.at[b], acc.at[didx.at[b]], ssem).wait()

    _fire(0)
    _fire(1)

    for r in range(0, RPT, ZR):
        pltpu.make_async_copy(zb0, acc.at[pl.ds(s * RPT + r, ZR)], osem).wait()

    @pl.when(s == NS - 1)
    def _():
        pltpu.make_async_copy(zb0.at[pl.ds(0, 16)], acc.at[pl.ds(NS * RPT, 16)],
                              osem).wait()

    plsc.subcore_barrier()

    @pl.loop(0, NCH)
    def _(j):
        _wait(j)
        _fire_s(j)

        @pl.when(j >= 1)
        def _():
            _wait_s(j - 1)

        @pl.when(j + 2 < NCH)
        def _():
            _fire(j + 2)

    _wait_s(NCH - 1)

    plsc.subcore_barrier()

    zbufs = (zb0, zb1)
    nch_out = RPT // ZR
    for r_i in range(nch_out):
        buf = zbufs[r_i % 2]
        lo = s * RPT + r_i * ZR
        if r_i >= 2:
            plo = s * RPT + (r_i - 2) * ZR
            pltpu.make_async_copy(buf, out_hbm.at[c, pl.ds(plo, ZR)], osem).wait()
        pltpu.sync_copy(acc.at[pl.ds(lo, ZR)], buf)
        pltpu.async_copy(buf, out_hbm.at[c, pl.ds(lo, ZR)], osem)
    for r_i in range(nch_out - 2, nch_out):
        buf = zbufs[r_i % 2]
        lo = s * RPT + r_i * ZR
        pltpu.make_async_copy(buf, out_hbm.at[c, pl.ds(lo, ZR)], osem).wait()

    @pl.when(s == NS - 1)
    def _():
        pltpu.sync_copy(acc.at[pl.ds(NS * RPT, 16)], zb0.at[pl.ds(0, 16)])
        pltpu.sync_copy(zb0.at[pl.ds(0, 16)], out_hbm.at[c, pl.ds(NS * RPT, 16)])


_sc_calls = {}


def _get_sc_calls():
    if not _sc_calls:
        mesh = plsc.VectorSubcoreMesh(core_axis_name="c", subcore_axis_name="s",
                                      num_cores=NC, num_subcores=NS)
        _sc_calls["deg"] = pl.kernel(
            _deg_body,
            out_type=jax.ShapeDtypeStruct((NC * N,), jnp.float32),
            mesh=mesh,
            scratch_types=[
                pltpu.VMEM_SHARED((N,), jnp.float32),
                pltpu.VMEM((NB, K), jnp.int32),
                pltpu.VMEM((K,), jnp.float32),
                pltpu.VMEM((2000,), jnp.float32),
                pltpu.SemaphoreType.DMA,
                pltpu.SemaphoreType.DMA,
            ],
        )
        _sc_calls["agg"] = pl.kernel(
            _agg_body,
            out_type=jax.ShapeDtypeStruct((NC, N, C), jnp.float32),
            mesh=mesh,
            scratch_types=[
                pltpu.VMEM_SHARED((N, C), jnp.float32),
                pltpu.VMEM((EPW,), jnp.int32),
                pltpu.VMEM((NB, K), jnp.int32),
                pltpu.VMEM((NB, K, C), jnp.float32),
                pltpu.VMEM((ZR, C), jnp.float32),
                pltpu.VMEM((ZR, C), jnp.float32),
                pltpu.SemaphoreType.DMA,
                pltpu.SemaphoreType.DMA,
                pltpu.SemaphoreType.DMA,
                pltpu.SemaphoreType.DMA,
            ],
        )
    return _sc_calls["deg"], _sc_calls["agg"]


def _tc1_body(x_ref, degp_ref, a1_ref, bnw_ref, bnb_ref, w1_ref, u1_ref, dinv_ref):
    x = x_ref[...]
    h = jnp.where(x >= 0, x, a1_ref[...] * x)
    mean = jnp.mean(h, axis=0, keepdims=True)
    var = jnp.mean(jnp.square(h - mean), axis=0, keepdims=True)
    h = (h - mean) * lax.rsqrt(var + 1e-5) * bnw_ref[...] + bnb_ref[...]
    deg = degp_ref[0] + degp_ref[1] + 1.0
    dinv = lax.rsqrt(deg)
    ht = jnp.dot(h, w1_ref[...], preferred_element_type=jnp.float32)
    u1_ref[...] = ht * dinv
    dinv_ref[...] = dinv


def _tc2_body(p_ref, u1_ref, dinv_ref, a2_ref, b1_ref, w2_ref, u2_ref):
    dinv = dinv_ref[...]
    agg = (p_ref[0] + p_ref[1] + u1_ref[...]) * dinv + b1_ref[...]
    h2 = jnp.where(agg >= 0, agg, a2_ref[...] * agg)
    u2_ref[...] = jnp.dot(h2, w2_ref[...], preferred_element_type=jnp.float32) * dinv


def _tc3_body(p_ref, u2_ref, dinv_ref, b2_ref, out_ref):
    out_ref[...] = (p_ref[0] + p_ref[1] + u2_ref[...]) * dinv_ref[...] + b2_ref[...]


_tc1 = pl.pallas_call(
    _tc1_body,
    out_shape=(jax.ShapeDtypeStruct((N, C), jnp.float32),
               jax.ShapeDtypeStruct((N, 1), jnp.float32)),
)

_tc2 = pl.pallas_call(
    _tc2_body,
    out_shape=jax.ShapeDtypeStruct((N, C), jnp.float32),
)

_tc3 = pl.pallas_call(
    _tc3_body,
    out_shape=jax.ShapeDtypeStruct((N, C), jnp.float32),
)


def kernel(x, edge_index, a1, bn_w, bn_b, W1, b1, a2, W2, b2):
    src = edge_index[0]
    dst = edge_index[1]
    _deg_call, _agg_call = _get_sc_calls()
    degp = _deg_call(dst)
    degp3 = degp.reshape(NC, N, 1)
    u1, dinv = _tc1(x, degp3, a1.reshape(1, 1), bn_w.reshape(1, C),
                    bn_b.reshape(1, C), W1)
    p1 = _agg_call(u1, src, dst)
    u2 = _tc2(p1, u1, dinv, a2.reshape(1, 1), b1.reshape(1, C), W2)
    p2 = _agg_call(u2, src, dst)
    out = _tc3(p2, u2, dinv, b2.reshape(1, C))
    return out


# speedup vs baseline: 33.9591x; 1.1084x over previous
import jax
import jax.numpy as jnp
from jax import lax
from jax.experimental import pallas as pl
from jax.experimental.pallas import tpu as pltpu
from jax.experimental.pallas import tpu_sc as plsc

N = 10000
C = 128
E = 320000
NC = 2
NS = 16
NW = NC * NS
K = 128
NCHT = E // K
NITER = (NCHT + NW - 1) // NW
NB = 3
RPT = 624
ZR = 104


def _deg_body(ei_hbm, out0_hbm, out1_hbm, acc, ebuf, ones_v, zbd, esem, ssem):
    c = lax.axis_index("c")
    s = lax.axis_index("s")
    w = c * NS + s

    @pl.loop(0, K, step=16)
    def _(i):
        ones_v.at[pl.ds(i, 16)][...] = jnp.ones((16,), jnp.float32)

    @pl.loop(0, 2000, step=16)
    def _(i):
        zbd.at[pl.ds(i, 16)][...] = jnp.zeros((16,), jnp.float32)

    @pl.when(s < 5)
    def _():
        pltpu.sync_copy(zbd, acc.at[pl.ds(s * 2000, 2000)])

    plsc.subcore_barrier()

    def _fire_e(j):
        cid = j * NW + w
        pltpu.async_copy(ei_hbm.at[:, pl.ds(cid * K, K)],
                         ebuf.at[lax.rem(j, NB)], esem)

    def _wait_e(j):
        cid = j * NW + w
        pltpu.make_async_copy(ei_hbm.at[:, pl.ds(cid * K, K)],
                              ebuf.at[lax.rem(j, NB)], esem).wait()

    def _fire_s(j):
        pltpu.async_copy(ones_v, acc.at[ebuf.at[lax.rem(j, NB), 1]], ssem,
                         add=True)

    def _wait_s(j):
        pltpu.make_async_copy(ones_v, acc.at[ebuf.at[lax.rem(j, NB), 1]],
                              ssem).wait()

    _fire_e(0)
    _fire_e(1)

    @pl.loop(0, NITER)
    def _(j):
        @pl.when(j * NW + w < NCHT)
        def _():
            _wait_e(j)
            _fire_s(j)

            @pl.when(j >= 1)
            def _():
                _wait_s(j - 1)

        @pl.when((j + 2) * NW + w < NCHT)
        def _():
            _fire_e(j + 2)

    @pl.when((NITER - 1) * NW + w < NCHT)
    def _():
        _wait_s(NITER - 1)

    @pl.when((NITER - 1) * NW + w >= NCHT)
    def _():
        _wait_s(NITER - 2)

    plsc.subcore_barrier()

    @pl.when(jnp.logical_and(c == 0, s < 10))
    def _():
        pltpu.sync_copy(acc.at[pl.ds(s * 1000, 1000)], zbd.at[pl.ds(0, 1000)])
        pltpu.sync_copy(zbd.at[pl.ds(0, 1000)], out0_hbm.at[pl.ds(s * 1000, 1000)])

    @pl.when(jnp.logical_and(c == 1, s < 10))
    def _():
        pltpu.sync_copy(acc.at[pl.ds(s * 1000, 1000)], zbd.at[pl.ds(0, 1000)])
        pltpu.sync_copy(zbd.at[pl.ds(0, 1000)], out1_hbm.at[pl.ds(s * 1000, 1000)])


def _agg_body(u_hbm, ei_hbm, out_hbm, acc, ebuf, rows, esem, gsem, ssem, osem):
    c = lax.axis_index("c")
    s = lax.axis_index("s")
    w = c * NS + s

    @pl.loop(0, ZR)
    def _(i):
        @pl.loop(0, C, step=16)
        def _(k):
            rows.at[2, pl.ds(i, 1), pl.ds(k, 16)][...] = jnp.zeros(
                (1, 16), jnp.float32)

    for r in range(0, RPT, ZR):
        pltpu.async_copy(rows.at[2, pl.ds(0, ZR)],
                         acc.at[pl.ds(s * RPT + r, ZR)], osem)

    @pl.when(s == NS - 1)
    def _():
        pltpu.async_copy(rows.at[2, pl.ds(0, 16)], acc.at[pl.ds(NS * RPT, 16)],
                         osem)

    def _fire_e(j):
        cid = j * NW + w
        pltpu.async_copy(ei_hbm.at[:, pl.ds(cid * K, K)],
                         ebuf.at[lax.rem(j, NB)], esem)

    def _wait_e(j):
        cid = j * NW + w
        pltpu.make_async_copy(ei_hbm.at[:, pl.ds(cid * K, K)],
                              ebuf.at[lax.rem(j, NB)], esem).wait()

    def _fire_g(j):
        b = lax.rem(j, NB)
        pltpu.async_copy(u_hbm.at[ebuf.at[b, 0]], rows.at[b], gsem)

    def _wait_g(j):
        b = lax.rem(j, NB)
        pltpu.make_async_copy(u_hbm.at[ebuf.at[b, 0]], rows.at[b], gsem).wait()

    def _fire_s(j):
        b = lax.rem(j, NB)
        pltpu.async_copy(rows.at[b], acc.at[ebuf.at[b, 1]], ssem, add=True)

    def _wait_s(j):
        b = lax.rem(j, NB)
        pltpu.make_async_copy(rows.at[b], acc.at[ebuf.at[b, 1]], ssem).wait()

    _fire_e(0)
    _fire_e(1)
    _wait_e(0)
    _fire_g(0)

    for r in range(0, RPT, ZR):
        pltpu.make_async_copy(rows.at[2, pl.ds(0, ZR)],
                              acc.at[pl.ds(s * RPT + r, ZR)], osem).wait()

    @pl.when(s == NS - 1)
    def _():
        pltpu.make_async_copy(rows.at[2, pl.ds(0, 16)],
                              acc.at[pl.ds(NS * RPT, 16)], osem).wait()

    plsc.subcore_barrier()

    @pl.loop(0, NITER)
    def _(j):
        @pl.when((j + 1) * NW + w < NCHT)
        def _():
            _wait_e(j + 1)
            _fire_g(j + 1)

        @pl.when(j * NW + w < NCHT)
        def _():
            _wait_g(j)
            _fire_s(j)

            @pl.when(j >= 1)
            def _():
                _wait_s(j - 1)

        @pl.when((j + 2) * NW + w < NCHT)
        def _():
            _fire_e(j + 2)

    @pl.when((NITER - 1) * NW + w < NCHT)
    def _():
        _wait_s(NITER - 1)

    @pl.when((NITER - 1) * NW + w >= NCHT)
    def _():
        _wait_s(NITER - 2)

    plsc.subcore_barrier()

    nch_out = RPT // ZR
    for r_i in range(nch_out):
        b = r_i % 2
        lo = s * RPT + r_i * ZR
        if r_i >= 2:
            plo = s * RPT + (r_i - 2) * ZR
            pltpu.make_async_copy(rows.at[b, pl.ds(0, ZR)],
                                  out_hbm.at[c, pl.ds(plo, ZR)], osem).wait()
        pltpu.sync_copy(acc.at[pl.ds(lo, ZR)], rows.at[b, pl.ds(0, ZR)])
        pltpu.async_copy(rows.at[b, pl.ds(0, ZR)],
                         out_hbm.at[c, pl.ds(lo, ZR)], osem)
    for r_i in range(nch_out - 2, nch_out):
        b = r_i % 2
        lo = s * RPT + r_i * ZR
        pltpu.make_async_copy(rows.at[b, pl.ds(0, ZR)],
                              out_hbm.at[c, pl.ds(lo, ZR)], osem).wait()

    @pl.when(s == NS - 1)
    def _():
        pltpu.sync_copy(acc.at[pl.ds(NS * RPT, 16)], rows.at[2, pl.ds(0, 16)])
        pltpu.sync_copy(rows.at[2, pl.ds(0, 16)],
                        out_hbm.at[c, pl.ds(NS * RPT, 16)])


_sc_calls = {}


def _get_sc_calls():
    if not _sc_calls:
        mesh = plsc.VectorSubcoreMesh(core_axis_name="c", subcore_axis_name="s",
                                      num_cores=NC, num_subcores=NS)
        _sc_calls["deg"] = pl.kernel(
            _deg_body,
            out_type=(jax.ShapeDtypeStruct((N,), jnp.float32),
                      jax.ShapeDtypeStruct((N,), jnp.float32)),
            mesh=mesh,
            scratch_types=[
                pltpu.VMEM_SHARED((N,), jnp.float32),
                pltpu.VMEM((NB, 2, K), jnp.int32),
                pltpu.VMEM((K,), jnp.float32),
                pltpu.VMEM((2000,), jnp.float32),
                pltpu.SemaphoreType.DMA,
                pltpu.SemaphoreType.DMA,
            ],
        )
        _sc_calls["agg"] = pl.kernel(
            _agg_body,
            out_type=jax.ShapeDtypeStruct((NC, N, C), jnp.float32),
            mesh=mesh,
            scratch_types=[
                pltpu.VMEM_SHARED((N, C), jnp.float32),
                pltpu.VMEM((NB, 2, K), jnp.int32),
                pltpu.VMEM((NB, K, C), jnp.float32),
                pltpu.SemaphoreType.DMA,
                pltpu.SemaphoreType.DMA,
                pltpu.SemaphoreType.DMA,
                pltpu.SemaphoreType.DMA,
            ],
        )
    return _sc_calls["deg"], _sc_calls["agg"]


def _tc1_body(x_ref, p0_ref, p1_ref, a1_ref, bnw_ref, bnb_ref, w1_ref,
              u1_ref, dinv_ref):
    x = x_ref[...]
    h = jnp.where(x >= 0, x, a1_ref[...] * x)
    mean = jnp.mean(h, axis=0, keepdims=True)
    var = jnp.mean(jnp.square(h - mean), axis=0, keepdims=True)
    h = (h - mean) * lax.rsqrt(var + 1e-5) * bnw_ref[...] + bnb_ref[...]
    deg = p0_ref[...] + p1_ref[...] + 1.0
    dinv = lax.rsqrt(deg).reshape(1, N)
    dinv_col = jnp.transpose(dinv, (1, 0))
    ht = jnp.dot(h, w1_ref[...], preferred_element_type=jnp.float32)
    u1_ref[...] = ht * dinv_col
    dinv_ref[...] = dinv_col


def _tc2_body(p_ref, u1_ref, dinv_ref, a2_ref, b1_ref, w2_ref, u2_ref):
    dinv = dinv_ref[...]
    agg = (p_ref[0] + p_ref[1] + u1_ref[...]) * dinv + b1_ref[...]
    h2 = jnp.where(agg >= 0, agg, a2_ref[...] * agg)
    u2_ref[...] = jnp.dot(h2, w2_ref[...], preferred_element_type=jnp.float32) * dinv


def _tc3_body(p_ref, u2_ref, dinv_ref, b2_ref, out_ref):
    out_ref[...] = (p_ref[0] + p_ref[1] + u2_ref[...]) * dinv_ref[...] + b2_ref[...]


_tc1 = pl.pallas_call(
    _tc1_body,
    out_shape=(jax.ShapeDtypeStruct((N, C), jnp.float32),
               jax.ShapeDtypeStruct((N, 1), jnp.float32)),
)

_tc2 = pl.pallas_call(
    _tc2_body,
    out_shape=jax.ShapeDtypeStruct((N, C), jnp.float32),
)

_tc3 = pl.pallas_call(
    _tc3_body,
    out_shape=jax.ShapeDtypeStruct((N, C), jnp.float32),
)


def kernel(x, edge_index, a1, bn_w, bn_b, W1, b1, a2, W2, b2):
    _deg_call, _agg_call = _get_sc_calls()
    p0, p1 = _deg_call(edge_index)
    u1, dinv = _tc1(x, p0, p1, a1.reshape(1, 1), bn_w.reshape(1, C),
                    bn_b.reshape(1, C), W1)
    pa1 = _agg_call(u1, edge_index)
    u2 = _tc2(pa1, u1, dinv, a2.reshape(1, 1), b1.reshape(1, C), W2)
    pa2 = _agg_call(u2, edge_index)
    out = _tc3(pa2, u2, dinv, b2.reshape(1, C))
    return out


# speedup vs baseline: 33.9930x; 1.0010x over previous
import jax
import jax.numpy as jnp
from jax import lax
from jax.experimental import pallas as pl
from jax.experimental.pallas import tpu as pltpu
from jax.experimental.pallas import tpu_sc as plsc

N = 10000
C = 128
E = 320000
NC = 2
NS = 16
NW = NC * NS
K = 128
NCHT = E // K
NITER = (NCHT + NW - 1) // NW
NB = 3
RPT = 624
ZR = 104


def _deg_body(ei_hbm, out0_hbm, out1_hbm, acc, ebuf, ones_v, zbd, esem, ssem):
    c = lax.axis_index("c")
    s = lax.axis_index("s")
    w = c * NS + s

    @pl.loop(0, K, step=16)
    def _(i):
        ones_v.at[pl.ds(i, 16)][...] = jnp.ones((16,), jnp.float32)

    @pl.loop(0, 2000, step=16)
    def _(i):
        zbd.at[pl.ds(i, 16)][...] = jnp.zeros((16,), jnp.float32)

    @pl.when(s < 5)
    def _():
        pltpu.sync_copy(zbd, acc.at[pl.ds(s * 2000, 2000)])

    plsc.subcore_barrier()

    def _fire_e(j):
        cid = j * NW + w
        pltpu.async_copy(ei_hbm.at[:, pl.ds(cid * K, K)],
                         ebuf.at[lax.rem(j, NB)], esem)

    def _wait_e(j):
        cid = j * NW + w
        pltpu.make_async_copy(ei_hbm.at[:, pl.ds(cid * K, K)],
                              ebuf.at[lax.rem(j, NB)], esem).wait()

    def _fire_s(j):
        pltpu.async_copy(ones_v, acc.at[ebuf.at[lax.rem(j, NB), 1]], ssem,
                         add=True)

    def _wait_s(j):
        pltpu.make_async_copy(ones_v, acc.at[ebuf.at[lax.rem(j, NB), 1]],
                              ssem).wait()

    _fire_e(0)
    _fire_e(1)

    @pl.loop(0, NITER)
    def _(j):
        @pl.when(j * NW + w < NCHT)
        def _():
            _wait_e(j)
            _fire_s(j)

            @pl.when(j >= 1)
            def _():
                _wait_s(j - 1)

        @pl.when((j + 2) * NW + w < NCHT)
        def _():
            _fire_e(j + 2)

    @pl.when((NITER - 1) * NW + w < NCHT)
    def _():
        _wait_s(NITER - 1)

    @pl.when((NITER - 1) * NW + w >= NCHT)
    def _():
        _wait_s(NITER - 2)

    plsc.subcore_barrier()

    @pl.when(jnp.logical_and(c == 0, s < 10))
    def _():
        pltpu.sync_copy(acc.at[pl.ds(s * 1000, 1000)], zbd.at[pl.ds(0, 1000)])
        pltpu.sync_copy(zbd.at[pl.ds(0, 1000)], out0_hbm.at[pl.ds(s * 1000, 1000)])

    @pl.when(jnp.logical_and(c == 1, s < 10))
    def _():
        pltpu.sync_copy(acc.at[pl.ds(s * 1000, 1000)], zbd.at[pl.ds(0, 1000)])
        pltpu.sync_copy(zbd.at[pl.ds(0, 1000)], out1_hbm.at[pl.ds(s * 1000, 1000)])


def _agg_body(u_hbm, ei_hbm, out_hbm, acc, ebuf, rows, esem, gsem, ssem, osem):
    c = lax.axis_index("c")
    s = lax.axis_index("s")
    w = c * NS + s

    @pl.loop(0, ZR)
    def _(i):
        @pl.loop(0, C, step=16)
        def _(k):
            rows.at[2, pl.ds(i, 1), pl.ds(k, 16)][...] = jnp.zeros(
                (1, 16), jnp.float32)

    for r in range(0, RPT, ZR):
        pltpu.async_copy(rows.at[2, pl.ds(0, ZR)],
                         acc.at[pl.ds(s * RPT + r, ZR)], osem)

    @pl.when(s == NS - 1)
    def _():
        pltpu.async_copy(rows.at[2, pl.ds(0, 16)], acc.at[pl.ds(NS * RPT, 16)],
                         osem)

    def _fire_e(j):
        cid = j * NW + w
        pltpu.async_copy(ei_hbm.at[:, pl.ds(cid * K, K)],
                         ebuf.at[lax.rem(j, NB)], esem)

    def _wait_e(j):
        cid = j * NW + w
        pltpu.make_async_copy(ei_hbm.at[:, pl.ds(cid * K, K)],
                              ebuf.at[lax.rem(j, NB)], esem).wait()

    def _fire_g(j):
        b = lax.rem(j, NB)
        pltpu.async_copy(u_hbm.at[ebuf.at[b, 0]], rows.at[b], gsem)

    def _wait_g(j):
        b = lax.rem(j, NB)
        pltpu.make_async_copy(u_hbm.at[ebuf.at[b, 0]], rows.at[b], gsem).wait()

    def _fire_s(j):
        b = lax.rem(j, NB)
        pltpu.async_copy(rows.at[b], acc.at[ebuf.at[b, 1]], ssem, add=True)

    def _wait_s(j):
        b = lax.rem(j, NB)
        pltpu.make_async_copy(rows.at[b], acc.at[ebuf.at[b, 1]], ssem).wait()

    _fire_e(0)
    _fire_e(1)
    _wait_e(0)
    _fire_g(0)

    for r in range(0, RPT, ZR):
        pltpu.make_async_copy(rows.at[2, pl.ds(0, ZR)],
                              acc.at[pl.ds(s * RPT + r, ZR)], osem).wait()

    @pl.when(s == NS - 1)
    def _():
        pltpu.make_async_copy(rows.at[2, pl.ds(0, 16)],
                              acc.at[pl.ds(NS * RPT, 16)], osem).wait()

    plsc.subcore_barrier()

    @pl.loop(0, NITER)
    def _(j):
        @pl.when((j + 1) * NW + w < NCHT)
        def _():
            _wait_e(j + 1)
            _fire_g(j + 1)

        @pl.when(j * NW + w < NCHT)
        def _():
            _wait_g(j)
            _fire_s(j)

            @pl.when(j >= 1)
            def _():
                _wait_s(j - 1)

        @pl.when((j + 2) * NW + w < NCHT)
        def _():
            _fire_e(j + 2)

    @pl.when((NITER - 1) * NW + w < NCHT)
    def _():
        _wait_s(NITER - 1)

    @pl.when((NITER - 1) * NW + w >= NCHT)
    def _():
        _wait_s(NITER - 2)

    plsc.subcore_barrier()

    nch_out = RPT // ZR
    for r_i in range(nch_out):
        b = r_i % 2
        lo = s * RPT + r_i * ZR
        if r_i >= 2:
            plo = s * RPT + (r_i - 2) * ZR
            pltpu.make_async_copy(rows.at[b, pl.ds(0, ZR)],
                                  out_hbm.at[c, pl.ds(plo, ZR)], osem).wait()
        pltpu.sync_copy(acc.at[pl.ds(lo, ZR)], rows.at[b, pl.ds(0, ZR)])
        pltpu.async_copy(rows.at[b, pl.ds(0, ZR)],
                         out_hbm.at[c, pl.ds(lo, ZR)], osem)
    for r_i in range(nch_out - 2, nch_out):
        b = r_i % 2
        lo = s * RPT + r_i * ZR
        pltpu.make_async_copy(rows.at[b, pl.ds(0, ZR)],
                              out_hbm.at[c, pl.ds(lo, ZR)], osem).wait()

    @pl.when(s == NS - 1)
    def _():
        pltpu.sync_copy(acc.at[pl.ds(NS * RPT, 16)], rows.at[2, pl.ds(0, 16)])
        pltpu.sync_copy(rows.at[2, pl.ds(0, 16)],
                        out_hbm.at[c, pl.ds(NS * RPT, 16)])


_sc_calls = {}


def _get_sc_calls():
    if not _sc_calls:
        mesh = plsc.VectorSubcoreMesh(core_axis_name="c", subcore_axis_name="s",
                                      num_cores=NC, num_subcores=NS)
        _sc_calls["deg"] = pl.kernel(
            _deg_body,
            out_type=(jax.ShapeDtypeStruct((N,), jnp.float32),
                      jax.ShapeDtypeStruct((N,), jnp.float32)),
            mesh=mesh,
            scratch_types=[
                pltpu.VMEM_SHARED((N,), jnp.float32),
                pltpu.VMEM((NB, 2, K), jnp.int32),
                pltpu.VMEM((K,), jnp.float32),
                pltpu.VMEM((2000,), jnp.float32),
                pltpu.SemaphoreType.DMA,
                pltpu.SemaphoreType.DMA,
            ],
        )
        _sc_calls["agg"] = pl.kernel(
            _agg_body,
            out_type=jax.ShapeDtypeStruct((NC, N, C), jnp.float32),
            mesh=mesh,
            scratch_types=[
                pltpu.VMEM_SHARED((N, C), jnp.float32),
                pltpu.VMEM((NB, 2, K), jnp.int32),
                pltpu.VMEM((NB, K, C), jnp.float32),
                pltpu.SemaphoreType.DMA,
                pltpu.SemaphoreType.DMA,
                pltpu.SemaphoreType.DMA,
                pltpu.SemaphoreType.DMA,
            ],
        )
    return _sc_calls["deg"], _sc_calls["agg"]


def _tc1a_body(x_ref, a1_ref, bnw_ref, bnb_ref, w1_ref, ht_ref):
    x = x_ref[...]
    h = jnp.where(x >= 0, x, a1_ref[...] * x)
    mean = jnp.mean(h, axis=0, keepdims=True)
    var = jnp.mean(jnp.square(h - mean), axis=0, keepdims=True)
    h = (h - mean) * lax.rsqrt(var + 1e-5) * bnw_ref[...] + bnb_ref[...]
    ht_ref[...] = jnp.dot(h, w1_ref[...], preferred_element_type=jnp.float32)


def _tc1b_body(ht_ref, p0_ref, p1_ref, u1_ref, dinv_ref):
    deg = p0_ref[...] + p1_ref[...] + 1.0
    dinv = lax.rsqrt(deg).reshape(1, N)
    dinv_col = jnp.transpose(dinv, (1, 0))
    u1_ref[...] = ht_ref[...] * dinv_col
    dinv_ref[...] = dinv_col


def _tc2_body(p_ref, u1_ref, dinv_ref, a2_ref, b1_ref, w2_ref, u2_ref):
    dinv = dinv_ref[...]
    agg = (p_ref[0] + p_ref[1] + u1_ref[...]) * dinv + b1_ref[...]
    h2 = jnp.where(agg >= 0, agg, a2_ref[...] * agg)
    u2_ref[...] = jnp.dot(h2, w2_ref[...], preferred_element_type=jnp.float32) * dinv


def _tc3_body(p_ref, u2_ref, dinv_ref, b2_ref, out_ref):
    out_ref[...] = (p_ref[0] + p_ref[1] + u2_ref[...]) * dinv_ref[...] + b2_ref[...]


_tc1a = pl.pallas_call(
    _tc1a_body,
    out_shape=jax.ShapeDtypeStruct((N, C), jnp.float32),
)

_tc1b = pl.pallas_call(
    _tc1b_body,
    out_shape=(jax.ShapeDtypeStruct((N, C), jnp.float32),
               jax.ShapeDtypeStruct((N, 1), jnp.float32)),
)

_tc2 = pl.pallas_call(
    _tc2_body,
    out_shape=jax.ShapeDtypeStruct((N, C), jnp.float32),
)

_tc3 = pl.pallas_call(
    _tc3_body,
    out_shape=jax.ShapeDtypeStruct((N, C), jnp.float32),
)


def kernel(x, edge_index, a1, bn_w, bn_b, W1, b1, a2, W2, b2):
    _deg_call, _agg_call = _get_sc_calls()
    p0, p1 = _deg_call(edge_index)
    ht = _tc1a(x, a1.reshape(1, 1), bn_w.reshape(1, C), bn_b.reshape(1, C), W1)
    u1, dinv = _tc1b(ht, p0, p1)
    pa1 = _agg_call(u1, edge_index)
    u2 = _tc2(pa1, u1, dinv, a2.reshape(1, 1), b1.reshape(1, C), W2)
    pa2 = _agg_call(u2, edge_index)
    out = _tc3(pa2, u2, dinv, b2.reshape(1, C))
    return out


# speedup vs baseline: 38.2440x; 1.1251x over previous
import jax
import jax.numpy as jnp
from jax import lax
from jax.experimental import pallas as pl
from jax.experimental.pallas import tpu as pltpu
from jax.experimental.pallas import tpu_sc as plsc

N = 10000
C = 128
E = 320000
NC = 2
NS = 16
NW = NC * NS
K = 128
NCHT = E // K
NITER = (NCHT + NW - 1) // NW
NB = 3
NBE = 4
RPT = 624
ZR = 104


def _deg_body(ei_hbm, out0_hbm, out1_hbm, acc, ebuf, ones_v, zbd, esem, ssem):
    c = lax.axis_index("c")
    s = lax.axis_index("s")
    w = c * NS + s

    @pl.loop(0, K, step=16)
    def _(i):
        ones_v.at[pl.ds(i, 16)][...] = jnp.ones((16,), jnp.float32)

    @pl.loop(0, 2000, step=16)
    def _(i):
        zbd.at[pl.ds(i, 16)][...] = jnp.zeros((16,), jnp.float32)

    @pl.when(s < 5)
    def _():
        pltpu.sync_copy(zbd, acc.at[pl.ds(s * 2000, 2000)])

    plsc.subcore_barrier()

    def _fire_e(j):
        cid = j * NW + w
        pltpu.async_copy(ei_hbm.at[:, pl.ds(cid * K, K)],
                         ebuf.at[lax.rem(j, NBE)], esem)

    def _wait_e(j):
        cid = j * NW + w
        pltpu.make_async_copy(ei_hbm.at[:, pl.ds(cid * K, K)],
                              ebuf.at[lax.rem(j, NBE)], esem).wait()

    def _fire_s(j):
        pltpu.async_copy(ones_v, acc.at[ebuf.at[lax.rem(j, NBE), 1]], ssem,
                         add=True)

    def _wait_s(j):
        pltpu.make_async_copy(ones_v, acc.at[ebuf.at[lax.rem(j, NBE), 1]],
                              ssem).wait()

    _fire_e(0)
    _fire_e(1)
    _fire_e(2)

    @pl.loop(0, NITER)
    def _(j):
        @pl.when(j * NW + w < NCHT)
        def _():
            _wait_e(j)
            _fire_s(j)

            @pl.when(j >= 1)
            def _():
                _wait_s(j - 1)

        @pl.when((j + 3) * NW + w < NCHT)
        def _():
            _fire_e(j + 3)

    @pl.when((NITER - 1) * NW + w < NCHT)
    def _():
        _wait_s(NITER - 1)

    @pl.when((NITER - 1) * NW + w >= NCHT)
    def _():
        _wait_s(NITER - 2)

    plsc.subcore_barrier()

    @pl.when(jnp.logical_and(c == 0, s < 10))
    def _():
        pltpu.sync_copy(acc.at[pl.ds(s * 1000, 1000)], zbd.at[pl.ds(0, 1000)])
        pltpu.sync_copy(zbd.at[pl.ds(0, 1000)], out0_hbm.at[pl.ds(s * 1000, 1000)])

    @pl.when(jnp.logical_and(c == 1, s < 10))
    def _():
        pltpu.sync_copy(acc.at[pl.ds(s * 1000, 1000)], zbd.at[pl.ds(0, 1000)])
        pltpu.sync_copy(zbd.at[pl.ds(0, 1000)], out1_hbm.at[pl.ds(s * 1000, 1000)])


def _agg_body(u_hbm, ei_hbm, out_hbm, acc, ebuf, rows, esem, gsem, ssem, osem):
    c = lax.axis_index("c")
    s = lax.axis_index("s")
    w = c * NS + s

    @pl.loop(0, ZR)
    def _(i):
        @pl.loop(0, C, step=16)
        def _(k):
            rows.at[2, pl.ds(i, 1), pl.ds(k, 16)][...] = jnp.zeros(
                (1, 16), jnp.float32)

    for r in range(0, RPT, ZR):
        pltpu.async_copy(rows.at[2, pl.ds(0, ZR)],
                         acc.at[pl.ds(s * RPT + r, ZR)], osem)

    @pl.when(s == NS - 1)
    def _():
        pltpu.async_copy(rows.at[2, pl.ds(0, 16)], acc.at[pl.ds(NS * RPT, 16)],
                         osem)

    def _fire_e(j):
        cid = j * NW + w
        pltpu.async_copy(ei_hbm.at[:, pl.ds(cid * K, K)],
                         ebuf.at[lax.rem(j, NBE)], esem)

    def _wait_e(j):
        cid = j * NW + w
        pltpu.make_async_copy(ei_hbm.at[:, pl.ds(cid * K, K)],
                              ebuf.at[lax.rem(j, NBE)], esem).wait()

    def _fire_g(j):
        be = lax.rem(j, NBE)
        pltpu.async_copy(u_hbm.at[ebuf.at[be, 0]], rows.at[lax.rem(j, NB)], gsem)

    def _wait_g(j):
        be = lax.rem(j, NBE)
        pltpu.make_async_copy(u_hbm.at[ebuf.at[be, 0]], rows.at[lax.rem(j, NB)],
                              gsem).wait()

    def _fire_s(j):
        be = lax.rem(j, NBE)
        pltpu.async_copy(rows.at[lax.rem(j, NB)], acc.at[ebuf.at[be, 1]], ssem,
                         add=True)

    def _wait_s(j):
        be = lax.rem(j, NBE)
        pltpu.make_async_copy(rows.at[lax.rem(j, NB)], acc.at[ebuf.at[be, 1]],
                              ssem).wait()

    _fire_e(0)
    _fire_e(1)
    _fire_e(2)
    _wait_e(0)
    _fire_g(0)

    for r in range(0, RPT, ZR):
        pltpu.make_async_copy(rows.at[2, pl.ds(0, ZR)],
                              acc.at[pl.ds(s * RPT + r, ZR)], osem).wait()

    @pl.when(s == NS - 1)
    def _():
        pltpu.make_async_copy(rows.at[2, pl.ds(0, 16)],
                              acc.at[pl.ds(NS * RPT, 16)], osem).wait()

    plsc.subcore_barrier()

    @pl.loop(0, NITER)
    def _(j):
        @pl.when((j + 1) * NW + w < NCHT)
        def _():
            _wait_e(j + 1)
            _fire_g(j + 1)

        @pl.when(j * NW + w < NCHT)
        def _():
            _wait_g(j)
            _fire_s(j)

            @pl.when(j >= 1)
            def _():
                _wait_s(j - 1)

        @pl.when((j + 3) * NW + w < NCHT)
        def _():
            _fire_e(j + 3)

    @pl.when((NITER - 1) * NW + w < NCHT)
    def _():
        _wait_s(NITER - 1)

    @pl.when((NITER - 1) * NW + w >= NCHT)
    def _():
        _wait_s(NITER - 2)

    plsc.subcore_barrier()

    nch_out = RPT // ZR
    for r_i in range(nch_out):
        b = r_i % 2
        lo = s * RPT + r_i * ZR
        if r_i >= 2:
            plo = s * RPT + (r_i - 2) * ZR
            pltpu.make_async_copy(rows.at[b, pl.ds(0, ZR)],
                                  out_hbm.at[c, pl.ds(plo, ZR)], osem).wait()
        pltpu.sync_copy(acc.at[pl.ds(lo, ZR)], rows.at[b, pl.ds(0, ZR)])
        pltpu.async_copy(rows.at[b, pl.ds(0, ZR)],
                         out_hbm.at[c, pl.ds(lo, ZR)], osem)
    for r_i in range(nch_out - 2, nch_out):
        b = r_i % 2
        lo = s * RPT + r_i * ZR
        pltpu.make_async_copy(rows.at[b, pl.ds(0, ZR)],
                              out_hbm.at[c, pl.ds(lo, ZR)], osem).wait()

    @pl.when(s == NS - 1)
    def _():
        pltpu.sync_copy(acc.at[pl.ds(NS * RPT, 16)], rows.at[2, pl.ds(0, 16)])
        pltpu.sync_copy(rows.at[2, pl.ds(0, 16)],
                        out_hbm.at[c, pl.ds(NS * RPT, 16)])


_sc_calls = {}


def _get_sc_calls():
    if not _sc_calls:
        mesh = plsc.VectorSubcoreMesh(core_axis_name="c", subcore_axis_name="s",
                                      num_cores=NC, num_subcores=NS)
        _sc_calls["deg"] = pl.kernel(
            _deg_body,
            out_type=(jax.ShapeDtypeStruct((N,), jnp.float32),
                      jax.ShapeDtypeStruct((N,), jnp.float32)),
            mesh=mesh,
            scratch_types=[
                pltpu.VMEM_SHARED((N,), jnp.float32),
                pltpu.VMEM((NBE, 2, K), jnp.int32),
                pltpu.VMEM((K,), jnp.float32),
                pltpu.VMEM((2000,), jnp.float32),
                pltpu.SemaphoreType.DMA,
                pltpu.SemaphoreType.DMA,
            ],
        )
        _sc_calls["agg"] = pl.kernel(
            _agg_body,
            out_type=jax.ShapeDtypeStruct((NC, N, C), jnp.float32),
            mesh=mesh,
            scratch_types=[
                pltpu.VMEM_SHARED((N, C), jnp.float32),
                pltpu.VMEM((NBE, 2, K), jnp.int32),
                pltpu.VMEM((NB, K, C), jnp.float32),
                pltpu.SemaphoreType.DMA,
                pltpu.SemaphoreType.DMA,
                pltpu.SemaphoreType.DMA,
                pltpu.SemaphoreType.DMA,
            ],
        )
    return _sc_calls["deg"], _sc_calls["agg"]


def _tc1a_body(x_ref, a1_ref, bnw_ref, bnb_ref, w1_ref, ht_ref):
    x = x_ref[...]
    h = jnp.where(x >= 0, x, a1_ref[...] * x)
    mean = jnp.mean(h, axis=0, keepdims=True)
    var = jnp.mean(jnp.square(h - mean), axis=0, keepdims=True)
    h = (h - mean) * lax.rsqrt(var + 1e-5) * bnw_ref[...] + bnb_ref[...]
    ht_ref[...] = jnp.dot(h, w1_ref[...], preferred_element_type=jnp.float32)


def _tc1b_body(ht_ref, p0_ref, p1_ref, u1_ref, dinv_ref):
    deg = p0_ref[...] + p1_ref[...] + 1.0
    dinv = lax.rsqrt(deg).reshape(1, N)
    dinv_col = jnp.transpose(dinv, (1, 0))
    u1_ref[...] = ht_ref[...] * dinv_col
    dinv_ref[...] = dinv_col


def _tc2_body(p_ref, u1_ref, dinv_ref, a2_ref, b1_ref, w2_ref, u2_ref):
    dinv = dinv_ref[...]
    agg = (p_ref[0] + p_ref[1] + u1_ref[...]) * dinv + b1_ref[...]
    h2 = jnp.where(agg >= 0, agg, a2_ref[...] * agg)
    u2_ref[...] = jnp.dot(h2, w2_ref[...], preferred_element_type=jnp.float32) * dinv


def _tc3_body(p_ref, u2_ref, dinv_ref, b2_ref, out_ref):
    out_ref[...] = (p_ref[0] + p_ref[1] + u2_ref[...]) * dinv_ref[...] + b2_ref[...]


_tc1a = pl.pallas_call(
    _tc1a_body,
    out_shape=jax.ShapeDtypeStruct((N, C), jnp.float32),
)

_tc1b = pl.pallas_call(
    _tc1b_body,
    out_shape=(jax.ShapeDtypeStruct((N, C), jnp.float32),
               jax.ShapeDtypeStruct((N, 1), jnp.float32)),
)

_tc2 = pl.pallas_call(
    _tc2_body,
    out_shape=jax.ShapeDtypeStruct((N, C), jnp.float32),
)

_tc3 = pl.pallas_call(
    _tc3_body,
    out_shape=jax.ShapeDtypeStruct((N, C), jnp.float32),
)


def kernel(x, edge_index, a1, bn_w, bn_b, W1, b1, a2, W2, b2):
    _deg_call, _agg_call = _get_sc_calls()
    p0, p1 = _deg_call(edge_index)
    ht = _tc1a(x, a1.reshape(1, 1), bn_w.reshape(1, C), bn_b.reshape(1, C), W1)
    u1, dinv = _tc1b(ht, p0, p1)
    pa1 = _agg_call(u1, edge_index)
    u2 = _tc2(pa1, u1, dinv, a2.reshape(1, 1), b1.reshape(1, C), W2)
    pa2 = _agg_call(u2, edge_index)
    out = _tc3(pa2, u2, dinv, b2.reshape(1, C))
    return out


# speedup vs baseline: 38.6989x; 1.0119x over previous
import jax
import jax.numpy as jnp
from jax import lax
from jax.experimental import pallas as pl
from jax.experimental.pallas import tpu as pltpu
from jax.experimental.pallas import tpu_sc as plsc

N = 10000
C = 128
E = 320000
NC = 2
NS = 16
NW = NC * NS
K = 128
NCHT = E // K
NITER = (NCHT + NW - 1) // NW
NB = 3
NBE = 5
RPT = 624
ZR = 104


def _deg_body(ei_hbm, out0_hbm, out1_hbm, acc, ebuf, ones_v, zbd, esem, ssem):
    c = lax.axis_index("c")
    s = lax.axis_index("s")
    w = c * NS + s

    @pl.loop(0, K, step=16)
    def _(i):
        ones_v.at[pl.ds(i, 16)][...] = jnp.ones((16,), jnp.float32)

    @pl.loop(0, 2000, step=16)
    def _(i):
        zbd.at[pl.ds(i, 16)][...] = jnp.zeros((16,), jnp.float32)

    @pl.when(s < 5)
    def _():
        pltpu.sync_copy(zbd, acc.at[pl.ds(s * 2000, 2000)])

    plsc.subcore_barrier()

    def _fire_e(j):
        cid = j * NW + w
        pltpu.async_copy(ei_hbm.at[:, pl.ds(cid * K, K)],
                         ebuf.at[lax.rem(j, NBE)], esem)

    def _wait_e(j):
        cid = j * NW + w
        pltpu.make_async_copy(ei_hbm.at[:, pl.ds(cid * K, K)],
                              ebuf.at[lax.rem(j, NBE)], esem).wait()

    def _fire_s(j):
        pltpu.async_copy(ones_v, acc.at[ebuf.at[lax.rem(j, NBE), 1]], ssem,
                         add=True)

    def _wait_s(j):
        pltpu.make_async_copy(ones_v, acc.at[ebuf.at[lax.rem(j, NBE), 1]],
                              ssem).wait()

    _fire_e(0)
    _fire_e(1)
    _fire_e(2)

    @pl.loop(0, NITER)
    def _(j):
        @pl.when(j * NW + w < NCHT)
        def _():
            _wait_e(j)
            _fire_s(j)

            @pl.when(j >= 1)
            def _():
                _wait_s(j - 1)

        @pl.when((j + 3) * NW + w < NCHT)
        def _():
            _fire_e(j + 3)

    @pl.when((NITER - 1) * NW + w < NCHT)
    def _():
        _wait_s(NITER - 1)

    @pl.when((NITER - 1) * NW + w >= NCHT)
    def _():
        _wait_s(NITER - 2)

    plsc.subcore_barrier()

    @pl.when(jnp.logical_and(c == 0, s < 10))
    def _():
        pltpu.sync_copy(acc.at[pl.ds(s * 1000, 1000)], zbd.at[pl.ds(0, 1000)])
        pltpu.sync_copy(zbd.at[pl.ds(0, 1000)], out0_hbm.at[pl.ds(s * 1000, 1000)])

    @pl.when(jnp.logical_and(c == 1, s < 10))
    def _():
        pltpu.sync_copy(acc.at[pl.ds(s * 1000, 1000)], zbd.at[pl.ds(0, 1000)])
        pltpu.sync_copy(zbd.at[pl.ds(0, 1000)], out1_hbm.at[pl.ds(s * 1000, 1000)])


def _agg_body(u_hbm, ei_hbm, out_hbm, acc, ebuf, rows, esem, gsem, ssem, osem):
    c = lax.axis_index("c")
    s = lax.axis_index("s")
    w = c * NS + s

    @pl.loop(0, ZR)
    def _(i):
        @pl.loop(0, C, step=16)
        def _(k):
            rows.at[2, pl.ds(i, 1), pl.ds(k, 16)][...] = jnp.zeros(
                (1, 16), jnp.float32)

    for r in range(0, RPT, ZR):
        pltpu.async_copy(rows.at[2, pl.ds(0, ZR)],
                         acc.at[pl.ds(s * RPT + r, ZR)], osem)

    @pl.when(s == NS - 1)
    def _():
        pltpu.async_copy(rows.at[2, pl.ds(0, 16)], acc.at[pl.ds(NS * RPT, 16)],
                         osem)

    def _fire_e(j):
        cid = j * NW + w
        pltpu.async_copy(ei_hbm.at[:, pl.ds(cid * K, K)],
                         ebuf.at[lax.rem(j, NBE)], esem)

    def _wait_e(j):
        cid = j * NW + w
        pltpu.make_async_copy(ei_hbm.at[:, pl.ds(cid * K, K)],
                              ebuf.at[lax.rem(j, NBE)], esem).wait()

    def _fire_g(j):
        be = lax.rem(j, NBE)
        pltpu.async_copy(u_hbm.at[ebuf.at[be, 0]], rows.at[lax.rem(j, NB)], gsem)

    def _wait_g(j):
        be = lax.rem(j, NBE)
        pltpu.make_async_copy(u_hbm.at[ebuf.at[be, 0]], rows.at[lax.rem(j, NB)],
                              gsem).wait()

    def _fire_s(j):
        be = lax.rem(j, NBE)
        pltpu.async_copy(rows.at[lax.rem(j, NB)], acc.at[ebuf.at[be, 1]], ssem,
                         add=True)

    def _wait_s(j):
        be = lax.rem(j, NBE)
        pltpu.make_async_copy(rows.at[lax.rem(j, NB)], acc.at[ebuf.at[be, 1]],
                              ssem).wait()

    _fire_e(0)
    _fire_e(1)
    _fire_e(2)
    _wait_e(0)
    _fire_g(0)
    _wait_e(1)
    _fire_g(1)

    for r in range(0, RPT, ZR):
        pltpu.make_async_copy(rows.at[2, pl.ds(0, ZR)],
                              acc.at[pl.ds(s * RPT + r, ZR)], osem).wait()

    @pl.when(s == NS - 1)
    def _():
        pltpu.make_async_copy(rows.at[2, pl.ds(0, 16)],
                              acc.at[pl.ds(NS * RPT, 16)], osem).wait()

    plsc.subcore_barrier()

    @pl.loop(0, NITER)
    def _(j):
        @pl.when(j * NW + w < NCHT)
        def _():
            _wait_g(j)
            _fire_s(j)

            @pl.when(j >= 1)
            def _():
                _wait_s(j - 1)

        @pl.when((j + 2) * NW + w < NCHT)
        def _():
            _wait_e(j + 2)
            _fire_g(j + 2)

        @pl.when((j + 3) * NW + w < NCHT)
        def _():
            _fire_e(j + 3)

    @pl.when((NITER - 1) * NW + w < NCHT)
    def _():
        _wait_s(NITER - 1)

    @pl.when((NITER - 1) * NW + w >= NCHT)
    def _():
        _wait_s(NITER - 2)

    plsc.subcore_barrier()

    nch_out = RPT // ZR
    for r_i in range(nch_out):
        b = r_i % 2
        lo = s * RPT + r_i * ZR
        if r_i >= 2:
            plo = s * RPT + (r_i - 2) * ZR
            pltpu.make_async_copy(rows.at[b, pl.ds(0, ZR)],
                                  out_hbm.at[c, pl.ds(plo, ZR)], osem).wait()
        pltpu.sync_copy(acc.at[pl.ds(lo, ZR)], rows.at[b, pl.ds(0, ZR)])
        pltpu.async_copy(rows.at[b, pl.ds(0, ZR)],
                         out_hbm.at[c, pl.ds(lo, ZR)], osem)
    for r_i in range(nch_out - 2, nch_out):
        b = r_i % 2
        lo = s * RPT + r_i * ZR
        pltpu.make_async_copy(rows.at[b, pl.ds(0, ZR)],
                              out_hbm.at[c, pl.ds(lo, ZR)], osem).wait()

    @pl.when(s == NS - 1)
    def _():
        pltpu.sync_copy(acc.at[pl.ds(NS * RPT, 16)], rows.at[2, pl.ds(0, 16)])
        pltpu.sync_copy(rows.at[2, pl.ds(0, 16)],
                        out_hbm.at[c, pl.ds(NS * RPT, 16)])


_sc_calls = {}


def _get_sc_calls():
    if not _sc_calls:
        mesh = plsc.VectorSubcoreMesh(core_axis_name="c", subcore_axis_name="s",
                                      num_cores=NC, num_subcores=NS)
        _sc_calls["deg"] = pl.kernel(
            _deg_body,
            out_type=(jax.ShapeDtypeStruct((N,), jnp.float32),
                      jax.ShapeDtypeStruct((N,), jnp.float32)),
            mesh=mesh,
            scratch_types=[
                pltpu.VMEM_SHARED((N,), jnp.float32),
                pltpu.VMEM((NBE, 2, K), jnp.int32),
                pltpu.VMEM((K,), jnp.float32),
                pltpu.VMEM((2000,), jnp.float32),
                pltpu.SemaphoreType.DMA,
                pltpu.SemaphoreType.DMA,
            ],
        )
        _sc_calls["agg"] = pl.kernel(
            _agg_body,
            out_type=jax.ShapeDtypeStruct((NC, N, C), jnp.float32),
            mesh=mesh,
            scratch_types=[
                pltpu.VMEM_SHARED((N, C), jnp.float32),
                pltpu.VMEM((NBE, 2, K), jnp.int32),
                pltpu.VMEM((NB, K, C), jnp.float32),
                pltpu.SemaphoreType.DMA,
                pltpu.SemaphoreType.DMA,
                pltpu.SemaphoreType.DMA,
                pltpu.SemaphoreType.DMA,
            ],
        )
    return _sc_calls["deg"], _sc_calls["agg"]


def _tc1a_body(x_ref, a1_ref, bnw_ref, bnb_ref, w1_ref, ht_ref):
    x = x_ref[...]
    h = jnp.where(x >= 0, x, a1_ref[...] * x)
    mean = jnp.mean(h, axis=0, keepdims=True)
    var = jnp.mean(jnp.square(h - mean), axis=0, keepdims=True)
    h = (h - mean) * lax.rsqrt(var + 1e-5) * bnw_ref[...] + bnb_ref[...]
    ht_ref[...] = jnp.dot(h, w1_ref[...], preferred_element_type=jnp.float32)


def _tc1b_body(ht_ref, p0_ref, p1_ref, u1_ref, dinv_ref):
    deg = p0_ref[...] + p1_ref[...] + 1.0
    dinv = lax.rsqrt(deg).reshape(1, N)
    dinv_col = jnp.transpose(dinv, (1, 0))
    u1_ref[...] = ht_ref[...] * dinv_col
    dinv_ref[...] = dinv_col


def _tc2_body(p_ref, u1_ref, dinv_ref, a2_ref, b1_ref, w2_ref, u2_ref):
    dinv = dinv_ref[...]
    agg = (p_ref[0] + p_ref[1] + u1_ref[...]) * dinv + b1_ref[...]
    h2 = jnp.where(agg >= 0, agg, a2_ref[...] * agg)
    u2_ref[...] = jnp.dot(h2, w2_ref[...], preferred_element_type=jnp.float32) * dinv


def _tc3_body(p_ref, u2_ref, dinv_ref, b2_ref, out_ref):
    out_ref[...] = (p_ref[0] + p_ref[1] + u2_ref[...]) * dinv_ref[...] + b2_ref[...]


_tc1a = pl.pallas_call(
    _tc1a_body,
    out_shape=jax.ShapeDtypeStruct((N, C), jnp.float32),
)

_tc1b = pl.pallas_call(
    _tc1b_body,
    out_shape=(jax.ShapeDtypeStruct((N, C), jnp.float32),
               jax.ShapeDtypeStruct((N, 1), jnp.float32)),
)

_tc2 = pl.pallas_call(
    _tc2_body,
    out_shape=jax.ShapeDtypeStruct((N, C), jnp.float32),
)

_tc3 = pl.pallas_call(
    _tc3_body,
    out_shape=jax.ShapeDtypeStruct((N, C), jnp.float32),
)


def kernel(x, edge_index, a1, bn_w, bn_b, W1, b1, a2, W2, b2):
    _deg_call, _agg_call = _get_sc_calls()
    p0, p1 = _deg_call(edge_index)
    ht = _tc1a(x, a1.reshape(1, 1), bn_w.reshape(1, C), bn_b.reshape(1, C), W1)
    u1, dinv = _tc1b(ht, p0, p1)
    pa1 = _agg_call(u1, edge_index)
    u2 = _tc2(pa1, u1, dinv, a2.reshape(1, 1), b1.reshape(1, C), W2)
    pa2 = _agg_call(u2, edge_index)
    out = _tc3(pa2, u2, dinv, b2.reshape(1, C))
    return out


# speedup vs baseline: 38.8773x; 1.0046x over previous
import jax
import jax.numpy as jnp
from jax import lax
from jax.experimental import pallas as pl
from jax.experimental.pallas import tpu as pltpu
from jax.experimental.pallas import tpu_sc as plsc

N = 10000
C = 128
E = 320000
NC = 2
NS = 16
NW = NC * NS
K = 128
NCHT = E // K
NITER = (NCHT + NW - 1) // NW
NB = 3
NBE = 6
RPT = 624
ZR = 104


def _deg_body(ei_hbm, out0_hbm, out1_hbm, acc, ebuf, ones_v, zbd, esem, ssem):
    c = lax.axis_index("c")
    s = lax.axis_index("s")
    w = c * NS + s

    @pl.loop(0, K, step=16)
    def _(i):
        ones_v.at[pl.ds(i, 16)][...] = jnp.ones((16,), jnp.float32)

    @pl.loop(0, 2000, step=16)
    def _(i):
        zbd.at[pl.ds(i, 16)][...] = jnp.zeros((16,), jnp.float32)

    @pl.when(s < 5)
    def _():
        pltpu.sync_copy(zbd, acc.at[pl.ds(s * 2000, 2000)])

    plsc.subcore_barrier()

    def _fire_e(j):
        cid = j * NW + w
        pltpu.async_copy(ei_hbm.at[:, pl.ds(cid * K, K)],
                         ebuf.at[lax.rem(j, NBE)], esem)

    def _wait_e(j):
        cid = j * NW + w
        pltpu.make_async_copy(ei_hbm.at[:, pl.ds(cid * K, K)],
                              ebuf.at[lax.rem(j, NBE)], esem).wait()

    def _fire_s(j):
        pltpu.async_copy(ones_v, acc.at[ebuf.at[lax.rem(j, NBE), 1]], ssem,
                         add=True)

    def _wait_s(j):
        pltpu.make_async_copy(ones_v, acc.at[ebuf.at[lax.rem(j, NBE), 1]],
                              ssem).wait()

    _fire_e(0)
    _fire_e(1)
    _fire_e(2)

    @pl.loop(0, NITER)
    def _(j):
        @pl.when(j * NW + w < NCHT)
        def _():
            _wait_e(j)
            _fire_s(j)

            @pl.when(j >= 1)
            def _():
                _wait_s(j - 1)

        @pl.when((j + 3) * NW + w < NCHT)
        def _():
            _fire_e(j + 3)

    @pl.when((NITER - 1) * NW + w < NCHT)
    def _():
        _wait_s(NITER - 1)

    @pl.when((NITER - 1) * NW + w >= NCHT)
    def _():
        _wait_s(NITER - 2)

    plsc.subcore_barrier()

    @pl.when(jnp.logical_and(c == 0, s < 10))
    def _():
        pltpu.sync_copy(acc.at[pl.ds(s * 1000, 1000)], zbd.at[pl.ds(0, 1000)])
        pltpu.sync_copy(zbd.at[pl.ds(0, 1000)], out0_hbm.at[pl.ds(s * 1000, 1000)])

    @pl.when(jnp.logical_and(c == 1, s < 10))
    def _():
        pltpu.sync_copy(acc.at[pl.ds(s * 1000, 1000)], zbd.at[pl.ds(0, 1000)])
        pltpu.sync_copy(zbd.at[pl.ds(0, 1000)], out1_hbm.at[pl.ds(s * 1000, 1000)])


def _agg_body(u_hbm, ei_hbm, out_hbm, acc, ebuf, rows, esem, gsem, ssem, osem):
    c = lax.axis_index("c")
    s = lax.axis_index("s")
    w = c * NS + s

    @pl.loop(0, ZR)
    def _(i):
        @pl.loop(0, C, step=16)
        def _(k):
            rows.at[2, pl.ds(i, 1), pl.ds(k, 16)][...] = jnp.zeros(
                (1, 16), jnp.float32)

    for r in range(0, RPT, ZR):
        pltpu.async_copy(rows.at[2, pl.ds(0, ZR)],
                         acc.at[pl.ds(s * RPT + r, ZR)], osem)

    @pl.when(s == NS - 1)
    def _():
        pltpu.async_copy(rows.at[2, pl.ds(0, 16)], acc.at[pl.ds(NS * RPT, 16)],
                         osem)

    def _fire_e(j):
        cid = j * NW + w
        pltpu.async_copy(ei_hbm.at[:, pl.ds(cid * K, K)],
                         ebuf.at[lax.rem(j, NBE)], esem)

    def _wait_e(j):
        cid = j * NW + w
        pltpu.make_async_copy(ei_hbm.at[:, pl.ds(cid * K, K)],
                              ebuf.at[lax.rem(j, NBE)], esem).wait()

    def _fire_g(j):
        be = lax.rem(j, NBE)
        pltpu.async_copy(u_hbm.at[ebuf.at[be, 0]], rows.at[lax.rem(j, NB)], gsem)

    def _wait_g(j):
        be = lax.rem(j, NBE)
        pltpu.make_async_copy(u_hbm.at[ebuf.at[be, 0]], rows.at[lax.rem(j, NB)],
                              gsem).wait()

    def _fire_s(j):
        be = lax.rem(j, NBE)
        pltpu.async_copy(rows.at[lax.rem(j, NB)], acc.at[ebuf.at[be, 1]], ssem,
                         add=True)

    def _wait_s(j):
        be = lax.rem(j, NBE)
        pltpu.make_async_copy(rows.at[lax.rem(j, NB)], acc.at[ebuf.at[be, 1]],
                              ssem).wait()

    _fire_e(0)
    _fire_e(1)
    _fire_e(2)
    _wait_e(0)
    _fire_g(0)
    _wait_e(1)
    _fire_g(1)

    for r in range(0, RPT, ZR):
        pltpu.make_async_copy(rows.at[2, pl.ds(0, ZR)],
                              acc.at[pl.ds(s * RPT + r, ZR)], osem).wait()

    @pl.when(s == NS - 1)
    def _():
        pltpu.make_async_copy(rows.at[2, pl.ds(0, 16)],
                              acc.at[pl.ds(NS * RPT, 16)], osem).wait()

    plsc.subcore_barrier()

    @pl.loop(0, NITER)
    def _(j):
        @pl.when(j * NW + w < NCHT)
        def _():
            _wait_g(j)
            _fire_s(j)

            @pl.when(j >= 1)
            def _():
                _wait_s(j - 1)

        @pl.when((j + 2) * NW + w < NCHT)
        def _():
            _wait_e(j + 2)
            _fire_g(j + 2)

        @pl.when((j + 3) * NW + w < NCHT)
        def _():
            _fire_e(j + 3)

    @pl.when((NITER - 1) * NW + w < NCHT)
    def _():
        _wait_s(NITER - 1)

    @pl.when((NITER - 1) * NW + w >= NCHT)
    def _():
        _wait_s(NITER - 2)

    plsc.subcore_barrier()

    nch_out = RPT // ZR
    for r_i in range(nch_out):
        b = r_i % 2
        lo = s * RPT + r_i * ZR
        if r_i >= 2:
            plo = s * RPT + (r_i - 2) * ZR
            pltpu.make_async_copy(rows.at[b, pl.ds(0, ZR)],
                                  out_hbm.at[c, pl.ds(plo, ZR)], osem).wait()
        pltpu.sync_copy(acc.at[pl.ds(lo, ZR)], rows.at[b, pl.ds(0, ZR)])
        pltpu.async_copy(rows.at[b, pl.ds(0, ZR)],
                         out_hbm.at[c, pl.ds(lo, ZR)], osem)
    for r_i in range(nch_out - 2, nch_out):
        b = r_i % 2
        lo = s * RPT + r_i * ZR
        pltpu.make_async_copy(rows.at[b, pl.ds(0, ZR)],
                              out_hbm.at[c, pl.ds(lo, ZR)], osem).wait()

    @pl.when(s == NS - 1)
    def _():
        pltpu.sync_copy(acc.at[pl.ds(NS * RPT, 16)], rows.at[2, pl.ds(0, 16)])
        pltpu.sync_copy(rows.at[2, pl.ds(0, 16)],
                        out_hbm.at[c, pl.ds(NS * RPT, 16)])


_sc_calls = {}


def _get_sc_calls():
    if not _sc_calls:
        mesh = plsc.VectorSubcoreMesh(core_axis_name="c", subcore_axis_name="s",
                                      num_cores=NC, num_subcores=NS)
        _sc_calls["deg"] = pl.kernel(
            _deg_body,
            out_type=(jax.ShapeDtypeStruct((N,), jnp.float32),
                      jax.ShapeDtypeStruct((N,), jnp.float32)),
            mesh=mesh,
            scratch_types=[
                pltpu.VMEM_SHARED((N,), jnp.float32),
                pltpu.VMEM((NBE, 2, K), jnp.int32),
                pltpu.VMEM((K,), jnp.float32),
                pltpu.VMEM((2000,), jnp.float32),
                pltpu.SemaphoreType.DMA,
                pltpu.SemaphoreType.DMA,
            ],
        )
        _sc_calls["agg"] = pl.kernel(
            _agg_body,
            out_type=jax.ShapeDtypeStruct((NC, N, C), jnp.float32),
            mesh=mesh,
            scratch_types=[
                pltpu.VMEM_SHARED((N, C), jnp.float32),
                pltpu.VMEM((NBE, 2, K), jnp.int32),
                pltpu.VMEM((NB, K, C), jnp.float32),
                pltpu.SemaphoreType.DMA,
                pltpu.SemaphoreType.DMA,
                pltpu.SemaphoreType.DMA,
                pltpu.SemaphoreType.DMA,
            ],
        )
    return _sc_calls["deg"], _sc_calls["agg"]


def _tc1a_body(x_ref, a1_ref, bnw_ref, bnb_ref, w1_ref, ht_ref):
    x = x_ref[...]
    h = jnp.where(x >= 0, x, a1_ref[...] * x)
    mean = jnp.mean(h, axis=0, keepdims=True)
    var = jnp.mean(jnp.square(h - mean), axis=0, keepdims=True)
    h = (h - mean) * lax.rsqrt(var + 1e-5) * bnw_ref[...] + bnb_ref[...]
    ht_ref[...] = jnp.dot(h, w1_ref[...], preferred_element_type=jnp.float32)


def _tc1b_body(ht_ref, p0_ref, p1_ref, u1_ref, dinv_ref):
    deg = p0_ref[...] + p1_ref[...] + 1.0
    dinv = lax.rsqrt(deg).reshape(1, N)
    dinv_col = jnp.transpose(dinv, (1, 0))
    u1_ref[...] = ht_ref[...] * dinv_col
    dinv_ref[...] = dinv


def _tc2_body(p_ref, u1_ref, dinv_ref, a2_ref, b1_ref, w2_ref, u2_ref):
    dinv = jnp.transpose(dinv_ref[...], (1, 0))
    agg = (p_ref[0] + p_ref[1] + u1_ref[...]) * dinv + b1_ref[...]
    h2 = jnp.where(agg >= 0, agg, a2_ref[...] * agg)
    u2_ref[...] = jnp.dot(h2, w2_ref[...], preferred_element_type=jnp.float32) * dinv


def _tc3_body(p_ref, u2_ref, dinv_ref, b2_ref, out_ref):
    dinv = jnp.transpose(dinv_ref[...], (1, 0))
    out_ref[...] = (p_ref[0] + p_ref[1] + u2_ref[...]) * dinv + b2_ref[...]


_tc1a = pl.pallas_call(
    _tc1a_body,
    out_shape=jax.ShapeDtypeStruct((N, C), jnp.float32),
)

_tc1b = pl.pallas_call(
    _tc1b_body,
    out_shape=(jax.ShapeDtypeStruct((N, C), jnp.float32),
               jax.ShapeDtypeStruct((1, N), jnp.float32)),
)

_tc2 = pl.pallas_call(
    _tc2_body,
    out_shape=jax.ShapeDtypeStruct((N, C), jnp.float32),
)

_tc3 = pl.pallas_call(
    _tc3_body,
    out_shape=jax.ShapeDtypeStruct((N, C), jnp.float32),
)


def kernel(x, edge_index, a1, bn_w, bn_b, W1, b1, a2, W2, b2):
    _deg_call, _agg_call = _get_sc_calls()
    p0, p1 = _deg_call(edge_index)
    ht = _tc1a(x, a1.reshape(1, 1), bn_w.reshape(1, C), bn_b.reshape(1, C), W1)
    u1, dinv = _tc1b(ht, p0, p1)
    pa1 = _agg_call(u1, edge_index)
    u2 = _tc2(pa1, u1, dinv, a2.reshape(1, 1), b1.reshape(1, C), W2)
    pa2 = _agg_call(u2, edge_index)
    out = _tc3(pa2, u2, dinv, b2.reshape(1, C))
    return out


# speedup vs baseline: 38.8987x; 1.0005x over previous
import jax
import jax.numpy as jnp
from jax import lax
from jax.experimental import pallas as pl
from jax.experimental.pallas import tpu as pltpu
from jax.experimental.pallas import tpu_sc as plsc

N = 10000
C = 128
E = 320000
NC = 2
NS = 16
NW = NC * NS
K = 128
NCHT = E // K
NITER = (NCHT + NW - 1) // NW
NB = 3
NBE = 6
RPT = 624
ZR = 104


def _deg_body(ei_hbm, out0_hbm, out1_hbm, acc, ebuf, ones_v, zbd, esem, ssem):
    c = lax.axis_index("c")
    s = lax.axis_index("s")
    w = c * NS + s

    @pl.loop(0, K, step=16)
    def _(i):
        ones_v.at[pl.ds(i, 16)][...] = jnp.ones((16,), jnp.float32)

    @pl.loop(0, 2000, step=16)
    def _(i):
        zbd.at[pl.ds(i, 16)][...] = jnp.zeros((16,), jnp.float32)

    @pl.when(s < 5)
    def _():
        pltpu.sync_copy(zbd, acc.at[pl.ds(s * 2000, 2000)])

    plsc.subcore_barrier()

    def _fire_e(j):
        cid = j * NW + w
        pltpu.async_copy(ei_hbm.at[:, pl.ds(cid * K, K)],
                         ebuf.at[lax.rem(j, NBE)], esem)

    def _wait_e(j):
        cid = j * NW + w
        pltpu.make_async_copy(ei_hbm.at[:, pl.ds(cid * K, K)],
                              ebuf.at[lax.rem(j, NBE)], esem).wait()

    def _fire_s(j):
        pltpu.async_copy(ones_v, acc.at[ebuf.at[lax.rem(j, NBE), 1]], ssem,
                         add=True)

    def _wait_s(j):
        pltpu.make_async_copy(ones_v, acc.at[ebuf.at[lax.rem(j, NBE), 1]],
                              ssem).wait()

    _fire_e(0)
    _fire_e(1)
    _fire_e(2)

    @pl.loop(0, NITER)
    def _(j):
        @pl.when(j * NW + w < NCHT)
        def _():
            _wait_e(j)
            _fire_s(j)

            @pl.when(j >= 3)
            def _():
                _wait_s(j - 3)

        @pl.when((j + 3) * NW + w < NCHT)
        def _():
            _fire_e(j + 3)

    for t in range(NITER - 4, NITER):
        @pl.when(jnp.logical_and(t * NW + w < NCHT,
                                 (t + 3) * NW + w >= NCHT))
        def _():
            _wait_s(t)

    plsc.subcore_barrier()

    @pl.when(jnp.logical_and(c == 0, s < 10))
    def _():
        pltpu.sync_copy(acc.at[pl.ds(s * 1000, 1000)], zbd.at[pl.ds(0, 1000)])
        pltpu.sync_copy(zbd.at[pl.ds(0, 1000)], out0_hbm.at[pl.ds(s * 1000, 1000)])

    @pl.when(jnp.logical_and(c == 1, s < 10))
    def _():
        pltpu.sync_copy(acc.at[pl.ds(s * 1000, 1000)], zbd.at[pl.ds(0, 1000)])
        pltpu.sync_copy(zbd.at[pl.ds(0, 1000)], out1_hbm.at[pl.ds(s * 1000, 1000)])


def _agg_body(u_hbm, ei_hbm, out_hbm, acc, ebuf, rows, esem, gsem, ssem, osem):
    c = lax.axis_index("c")
    s = lax.axis_index("s")
    w = c * NS + s

    @pl.loop(0, ZR)
    def _(i):
        @pl.loop(0, C, step=16)
        def _(k):
            rows.at[2, pl.ds(i, 1), pl.ds(k, 16)][...] = jnp.zeros(
                (1, 16), jnp.float32)

    for r in range(0, RPT, ZR):
        pltpu.async_copy(rows.at[2, pl.ds(0, ZR)],
                         acc.at[pl.ds(s * RPT + r, ZR)], osem)

    @pl.when(s == NS - 1)
    def _():
        pltpu.async_copy(rows.at[2, pl.ds(0, 16)], acc.at[pl.ds(NS * RPT, 16)],
                         osem)

    def _fire_e(j):
        cid = j * NW + w
        pltpu.async_copy(ei_hbm.at[:, pl.ds(cid * K, K)],
                         ebuf.at[lax.rem(j, NBE)], esem)

    def _wait_e(j):
        cid = j * NW + w
        pltpu.make_async_copy(ei_hbm.at[:, pl.ds(cid * K, K)],
                              ebuf.at[lax.rem(j, NBE)], esem).wait()

    def _fire_g(j):
        be = lax.rem(j, NBE)
        pltpu.async_copy(u_hbm.at[ebuf.at[be, 0]], rows.at[lax.rem(j, NB)], gsem)

    def _wait_g(j):
        be = lax.rem(j, NBE)
        pltpu.make_async_copy(u_hbm.at[ebuf.at[be, 0]], rows.at[lax.rem(j, NB)],
                              gsem).wait()

    def _fire_s(j):
        be = lax.rem(j, NBE)
        pltpu.async_copy(rows.at[lax.rem(j, NB)], acc.at[ebuf.at[be, 1]], ssem,
                         add=True)

    def _wait_s(j):
        be = lax.rem(j, NBE)
        pltpu.make_async_copy(rows.at[lax.rem(j, NB)], acc.at[ebuf.at[be, 1]],
                              ssem).wait()

    _fire_e(0)
    _fire_e(1)
    _fire_e(2)
    _wait_e(0)
    _fire_g(0)
    _wait_e(1)
    _fire_g(1)

    for r in range(0, RPT, ZR):
        pltpu.make_async_copy(rows.at[2, pl.ds(0, ZR)],
                              acc.at[pl.ds(s * RPT + r, ZR)], osem).wait()

    @pl.when(s == NS - 1)
    def _():
        pltpu.make_async_copy(rows.at[2, pl.ds(0, 16)],
                              acc.at[pl.ds(NS * RPT, 16)], osem).wait()

    plsc.subcore_barrier()

    @pl.loop(0, NITER)
    def _(j):
        @pl.when(j * NW + w < NCHT)
        def _():
            _wait_g(j)
            _fire_s(j)

            @pl.when(j >= 1)
            def _():
                _wait_s(j - 1)

        @pl.when((j + 2) * NW + w < NCHT)
        def _():
            _wait_e(j + 2)
            _fire_g(j + 2)

        @pl.when((j + 3) * NW + w < NCHT)
        def _():
            _fire_e(j + 3)

    @pl.when((NITER - 1) * NW + w < NCHT)
    def _():
        _wait_s(NITER - 1)

    @pl.when((NITER - 1) * NW + w >= NCHT)
    def _():
        _wait_s(NITER - 2)

    plsc.subcore_barrier()

    nch_out = RPT // ZR
    for r_i in range(nch_out):
        b = r_i % 2
        lo = s * RPT + r_i * ZR
        if r_i >= 2:
            plo = s * RPT + (r_i - 2) * ZR
            pltpu.make_async_copy(rows.at[b, pl.ds(0, ZR)],
                                  out_hbm.at[c, pl.ds(plo, ZR)], osem).wait()
        pltpu.sync_copy(acc.at[pl.ds(lo, ZR)], rows.at[b, pl.ds(0, ZR)])
        pltpu.async_copy(rows.at[b, pl.ds(0, ZR)],
                         out_hbm.at[c, pl.ds(lo, ZR)], osem)
    for r_i in range(nch_out - 2, nch_out):
        b = r_i % 2
        lo = s * RPT + r_i * ZR
        pltpu.make_async_copy(rows.at[b, pl.ds(0, ZR)],
                              out_hbm.at[c, pl.ds(lo, ZR)], osem).wait()

    @pl.when(s == NS - 1)
    def _():
        pltpu.sync_copy(acc.at[pl.ds(NS * RPT, 16)], rows.at[2, pl.ds(0, 16)])
        pltpu.sync_copy(rows.at[2, pl.ds(0, 16)],
                        out_hbm.at[c, pl.ds(NS * RPT, 16)])


_sc_calls = {}


def _get_sc_calls():
    if not _sc_calls:
        mesh = plsc.VectorSubcoreMesh(core_axis_name="c", subcore_axis_name="s",
                                      num_cores=NC, num_subcores=NS)
        _sc_calls["deg"] = pl.kernel(
            _deg_body,
            out_type=(jax.ShapeDtypeStruct((N,), jnp.float32),
                      jax.ShapeDtypeStruct((N,), jnp.float32)),
            mesh=mesh,
            scratch_types=[
                pltpu.VMEM_SHARED((N,), jnp.float32),
                pltpu.VMEM((NBE, 2, K), jnp.int32),
                pltpu.VMEM((K,), jnp.float32),
                pltpu.VMEM((2000,), jnp.float32),
                pltpu.SemaphoreType.DMA,
                pltpu.SemaphoreType.DMA,
            ],
        )
        _sc_calls["agg"] = pl.kernel(
            _agg_body,
            out_type=jax.ShapeDtypeStruct((NC, N, C), jnp.float32),
            mesh=mesh,
            scratch_types=[
                pltpu.VMEM_SHARED((N, C), jnp.float32),
                pltpu.VMEM((NBE, 2, K), jnp.int32),
                pltpu.VMEM((NB, K, C), jnp.float32),
                pltpu.SemaphoreType.DMA,
                pltpu.SemaphoreType.DMA,
                pltpu.SemaphoreType.DMA,
                pltpu.SemaphoreType.DMA,
            ],
        )
    return _sc_calls["deg"], _sc_calls["agg"]


def _tc1a_body(x_ref, a1_ref, bnw_ref, bnb_ref, w1_ref, ht_ref):
    x = x_ref[...]
    h = jnp.where(x >= 0, x, a1_ref[...] * x)
    mean = jnp.mean(h, axis=0, keepdims=True)
    var = jnp.mean(jnp.square(h - mean), axis=0, keepdims=True)
    h = (h - mean) * lax.rsqrt(var + 1e-5) * bnw_ref[...] + bnb_ref[...]
    ht_ref[...] = jnp.dot(h, w1_ref[...], preferred_element_type=jnp.float32)


def _tc1b_body(ht_ref, p0_ref, p1_ref, u1_ref, dinv_ref):
    deg = p0_ref[...] + p1_ref[...] + 1.0
    dinv = lax.rsqrt(deg).reshape(1, N)
    dinv_col = jnp.transpose(dinv, (1, 0))
    u1_ref[...] = ht_ref[...] * dinv_col
    dinv_ref[...] = dinv


def _tc2_body(p_ref, u1_ref, dinv_ref, a2_ref, b1_ref, w2_ref, u2_ref):
    dinv = jnp.transpose(dinv_ref[...], (1, 0))
    agg = (p_ref[0] + p_ref[1] + u1_ref[...]) * dinv + b1_ref[...]
    h2 = jnp.where(agg >= 0, agg, a2_ref[...] * agg)
    u2_ref[...] = jnp.dot(h2, w2_ref[...], preferred_element_type=jnp.float32) * dinv


def _tc3_body(p_ref, u2_ref, dinv_ref, b2_ref, out_ref):
    dinv = jnp.transpose(dinv_ref[...], (1, 0))
    out_ref[...] = (p_ref[0] + p_ref[1] + u2_ref[...]) * dinv + b2_ref[...]


_tc1a = pl.pallas_call(
    _tc1a_body,
    out_shape=jax.ShapeDtypeStruct((N, C), jnp.float32),
)

_tc1b = pl.pallas_call(
    _tc1b_body,
    out_shape=(jax.ShapeDtypeStruct((N, C), jnp.float32),
               jax.ShapeDtypeStruct((1, N), jnp.float32)),
)

_tc2 = pl.pallas_call(
    _tc2_body,
    out_shape=jax.ShapeDtypeStruct((N, C), jnp.float32),
)

_tc3 = pl.pallas_call(
    _tc3_body,
    out_shape=jax.ShapeDtypeStruct((N, C), jnp.float32),
)


def kernel(x, edge_index, a1, bn_w, bn_b, W1, b1, a2, W2, b2):
    _deg_call, _agg_call = _get_sc_calls()
    p0, p1 = _deg_call(edge_index)
    ht = _tc1a(x, a1.reshape(1, 1), bn_w.reshape(1, C), bn_b.reshape(1, C), W1)
    u1, dinv = _tc1b(ht, p0, p1)
    pa1 = _agg_call(u1, edge_index)
    u2 = _tc2(pa1, u1, dinv, a2.reshape(1, 1), b1.reshape(1, C), W2)
    pa2 = _agg_call(u2, edge_index)
    out = _tc3(pa2, u2, dinv, b2.reshape(1, C))
    return out
